# baseline (speedup 1.0000x reference)
import jax
import jax.numpy as jnp
from jax.experimental import pallas as pl

TRANS_DIM = 384
ENC_DIM = 384
NUM_GROUP = 128
GROUP_SIZE = 32
GRID_SIZE = 0.02
BITS = 10


def _bn(x, g, b):
    mu = x.mean(axis=(0, 1), keepdims=True)
    var = x.var(axis=(0, 1), keepdims=True)
    return (x - mu) / jnp.sqrt(var + 1e-5) * g + b


def _fps(xyz, K):
    B, N, _ = xyz.shape
    dists = jnp.full((B, N), 1e10, dtype=xyz.dtype)
    farthest = jnp.zeros((B,), dtype=jnp.int32)
    idx_list = []
    for _ in range(K):
        idx_list.append(farthest)
        centroid = jnp.take_along_axis(xyz, farthest[:, None, None], axis=1)
        d = jnp.sum((xyz - centroid) ** 2, axis=-1)
        dists = jnp.minimum(dists, d)
        farthest = jnp.argmax(dists, axis=1).astype(jnp.int32)
    idxs = jnp.stack(idx_list, axis=1)
    centers = jnp.take_along_axis(xyz, idxs[:, :, None], axis=1)
    return centers, idxs


def _knn_idx(center, xyz, K):
    d2 = jnp.sum(center ** 2, -1)[:, :, None] + jnp.sum(xyz ** 2, -1)[:, None, :] - 2.0 * jnp.einsum('bgc,bnc->bgn', center, xyz)
    _, idx = jax.lax.top_k(-d2, K)
    return idx


def _morton(grid, perm):
    x = grid[..., perm[0]]
    y = grid[..., perm[1]]
    z = grid[..., perm[2]]
    code = jnp.zeros(x.shape, dtype=jnp.int32)
    for b in range(BITS):
        code = code | (((x >> b) & 1) << (3 * b + 2)) | (((y >> b) & 1) << (3 * b + 1)) | (((z >> b) & 1) << (3 * b))
    return code


def _serialize(pos, feats, perm):
    grid = jnp.floor(pos / GRID_SIZE).astype(jnp.int32)
    grid = grid - grid.min(axis=1, keepdims=True)
    grid = jnp.clip(grid, 0, (1 << BITS) - 1)
    code = _morton(grid, perm)
    order = jnp.argsort(code, axis=1)
    return [jnp.take_along_axis(f, order[:, :, None], axis=1) for f in feats]


def _final_pallas(x, pos_all):
    B, T, D = x.shape

    def body(x_ref, p_ref, o_ref):
        xs = x_ref[0] + p_ref[0]
        cls = xs[T - 1, :]
        mean = jnp.mean(xs, axis=0)
        o_ref[0, 0] = jnp.concatenate([cls, mean], axis=-1)

    out = pl.pallas_call(
        body,
        grid=(B,),
        in_specs=[
            pl.BlockSpec((1, T, D), lambda b: (b, 0, 0)),
            pl.BlockSpec((1, T, D), lambda b: (b, 0, 0)),
        ],
        out_specs=pl.BlockSpec((1, 1, 2 * D), lambda b: (b, 0, 0)),
        out_shape=jax.ShapeDtypeStruct((B, 1, 2 * D), x.dtype),
    )(x, pos_all)
    return out.reshape(B, 2 * D)


def kernel(pts, W1, b1, g1, be1, W2, b2, W3, b3, g2, be2, W4, b4, Wp1, bp1, Wp2, bp2, gamma1, beta1, gamma2, beta2, cls_token, cls_pos):
    B, N, _ = pts.shape
    centers, _ = _fps(pts, NUM_GROUP)
    idx = _knn_idx(centers, pts, GROUP_SIZE)
    neigh = jnp.take_along_axis(pts, idx.reshape(B, -1)[:, :, None], axis=1).reshape(B, NUM_GROUP, GROUP_SIZE, 3)
    neigh = neigh - centers[:, :, None, :]
    pg = neigh.reshape(B * NUM_GROUP, GROUP_SIZE, 3)
    f = pg @ W1.T + b1
    f = jax.nn.relu(_bn(f, g1, be1))
    f = f @ W2.T + b2
    fg = f.max(axis=1, keepdims=True)
    f = jnp.concatenate([jnp.broadcast_to(fg, f.shape), f], axis=-1)
    f = f @ W3.T + b3
    f = jax.nn.relu(_bn(f, g2, be2))
    f = f @ W4.T + b4
    tokens = f.max(axis=1).reshape(B, NUM_GROUP, ENC_DIM)
    pos = jax.nn.gelu(centers @ Wp1.T + bp1) @ Wp2.T + bp2
    tok_f, pos_f = _serialize(centers, [tokens, pos], (0, 1, 2))
    tok_b, pos_b = _serialize(centers, [tokens, pos], (2, 1, 0))
    tok_f = tok_f * gamma1 + beta1
    tok_b = tok_b * gamma2 + beta2
    cls_t = jnp.broadcast_to(cls_token, (B, 1, TRANS_DIM))
    cls_p = jnp.broadcast_to(cls_pos, (B, 1, TRANS_DIM))
    pos_all = jnp.concatenate([pos_f, pos_b, cls_t], axis=1)
    x = jnp.concatenate([tok_f, tok_b, cls_p], axis=1)
    return _final_pallas(x, pos_all)


# speedup vs baseline: 3.7336x; 3.7336x over previous
import functools

import jax
import jax.numpy as jnp
from jax import lax
from jax.experimental import pallas as pl
from jax.experimental.pallas import tpu as pltpu
from jax.experimental.pallas import tpu_sc as plsc

TRANS_DIM = 384
ENC_DIM = 384
NUM_GROUP = 128
GROUP_SIZE = 32
GRID_SIZE = 0.02
BITS = 10

B = 32
N = 8192
NROWS = B * NUM_GROUP
CAND_CAP = 4096


def _k2_body(pts_ref, c_ref, d2_ref, t0_ref):
    pts_mat = pts_ref[:, 0, 0, :]
    c_mat = c_ref[:, 0, 0, :]
    x2 = pts_mat * pts_mat
    pp = (x2[0:1] + x2[1:2]) + x2[2:3]
    c2 = c_mat * c_mat
    cc = (c2[0:1] + c2[1:2]) + c2[2:3]
    prod = jax.lax.dot_general(
        c_mat, pts_mat, (((0,), (0,)), ((), ())),
        preferred_element_type=jnp.float32)
    d2 = (jnp.transpose(cc) + pp) - 2.0 * prod
    d2_ref[...] = d2
    m = jnp.min(d2.reshape(NUM_GROUP, 32, N // 32), axis=2)
    t0_ref[0, 0] = jnp.max(m, axis=1)


def _k2(ptsT, cxyz):
    d2, t0 = pl.pallas_call(
        _k2_body,
        grid=(B,),
        in_specs=[
            pl.BlockSpec((3, 1, 1, N), lambda b: (0, b, 0, 0)),
            pl.BlockSpec((3, 1, 1, NUM_GROUP), lambda b: (0, b, 0, 0)),
        ],
        out_specs=[
            pl.BlockSpec((NUM_GROUP, N), lambda b: (b, 0)),
            pl.BlockSpec((1, 1, NUM_GROUP), lambda b: (b, 0, 0)),
        ],
        out_shape=[
            jax.ShapeDtypeStruct((NROWS, N), jnp.float32),
            jax.ShapeDtypeStruct((B, 1, NUM_GROUP), jnp.float32),
        ],
    )(ptsT.reshape(3, B, 1, N), cxyz.reshape(3, B, 1, NUM_GROUP))
    return d2, t0.reshape(B, NUM_GROUP)


def _k3_sc(d2, ptsT, cxyz, t0):
    info = plsc.get_sparse_core_info()
    NC = info.num_cores

    mesh = plsc.VectorSubcoreMesh(core_axis_name="c", subcore_axis_name="s")
    G = NUM_GROUP
    K = GROUP_SIZE

    @functools.partial(
        pl.kernel,
        mesh=mesh,
        compiler_params=pltpu.CompilerParams(needs_layout_passes=False),
        out_type=[
            jax.ShapeDtypeStruct((B * G * K * 3,), jnp.float32),
            jax.ShapeDtypeStruct((B, 144), jnp.float32),
        ],
        scratch_types=[
            pltpu.VMEM((N,), jnp.float32),
            pltpu.VMEM((N,), jnp.float32),
            pltpu.VMEM((N,), jnp.float32),
            pltpu.VMEM((G,), jnp.float32),
            pltpu.VMEM((G,), jnp.float32),
            pltpu.VMEM((G,), jnp.float32),
            pltpu.VMEM((G,), jnp.float32),
            pltpu.VMEM((N,), jnp.float32),
            pltpu.VMEM((CAND_CAP,), jnp.float32),
            pltpu.VMEM((CAND_CAP,), jnp.int32),
            pltpu.VMEM((64,), jnp.int32),
            pltpu.VMEM((G * K * 3,), jnp.float32),
            pltpu.VMEM((144,), jnp.float32),
        ],
    )
    def k3(d2_hbm, ptsT_hbm, cxyz_hbm, t0_hbm, pg_out, mom_out,
           xb, yb, zb, cxb, cyb, czb, t0b, drow, cand_v, cand_i,
           sel_idx, pgblock, mombuf):
        b = lax.axis_index("s") * NC + lax.axis_index("c")

        pltpu.sync_copy(ptsT_hbm.at[0, b], xb)
        pltpu.sync_copy(ptsT_hbm.at[1, b], yb)
        pltpu.sync_copy(ptsT_hbm.at[2, b], zb)
        pltpu.sync_copy(cxyz_hbm.at[0, b], cxb)
        pltpu.sync_copy(cxyz_hbm.at[1, b], cyb)
        pltpu.sync_copy(cxyz_hbm.at[2, b], czb)
        pltpu.sync_copy(t0_hbm.at[b], t0b)

        iota = lax.iota(jnp.int32, 16)
        zero16f = jnp.zeros((16,), jnp.float32)

        def row_body(r, moms):
            pltpu.sync_copy(d2_hbm.at[b * G + r], drow)
            rsp = jnp.full((16,), r, jnp.int32)
            t0s = plsc.load_gather(t0b, [rsp])
            cxs = plsc.load_gather(cxb, [rsp])
            cys = plsc.load_gather(cyb, [rsp])
            czs = plsc.load_gather(czb, [rsp])

            def scan_body(j, off):
                v = drow[pl.ds(j * 16, 16)]
                m = v <= t0s
                mi = m.astype(jnp.int32)
                pos = off + plsc.cumsum(mi) - 1
                pos = jnp.minimum(pos, CAND_CAP - 1)
                idxv = iota + j * 16
                plsc.store_scatter(cand_v, [pos], v, mask=m)
                plsc.store_scatter(cand_i, [pos], idxv, mask=m)
                return off + plsc.all_reduce_population_count(m)

            cnt_sp = lax.fori_loop(0, N // 16, scan_body,
                                   jnp.zeros((16,), jnp.int32))
            cnt = jnp.max(cnt_sp)
            nv = (cnt + 15) // 16

            def count_le(mid_sp):
                def cbody(c, acc):
                    vbits = plsc.bitcast(cand_v[pl.ds(c * 16, 16)], jnp.int32)
                    valid = (iota + c * 16) < cnt_sp
                    le = jnp.logical_and(vbits <= mid_sp, valid)
                    return acc + plsc.all_reduce_population_count(le)
                return lax.fori_loop(0, nv, cbody, jnp.zeros((16,), jnp.int32))

            def bis_body(_, st):
                lo, hi, cnt_lo = st
                mid = jnp.right_shift(lo + hi, 1)
                c = count_le(mid)
                ge = c >= 32
                hi = jnp.where(ge, mid, hi)
                new_lo = jnp.where(ge, lo, mid)
                cnt_lo = jnp.where(ge, cnt_lo, c)
                return new_lo, hi, cnt_lo

            lo0 = jnp.full((16,), jnp.int32(-2147483647), jnp.int32)
            hi0 = jnp.full((16,), jnp.int32(0x7F800000), jnp.int32)
            lo, hi, cnt_lo = lax.fori_loop(
                0, 32, bis_body, (lo0, hi0, jnp.zeros((16,), jnp.int32)))
            t32 = hi
            need_eq = 32 - cnt_lo

            def emit_body(c, st):
                soff, eqoff = st
                vbits = plsc.bitcast(cand_v[pl.ds(c * 16, 16)], jnp.int32)
                ci = cand_i[pl.ds(c * 16, 16)]
                valid = (iota + c * 16) < cnt_sp
                m_lt = jnp.logical_and(vbits < t32, valid)
                m_eq = jnp.logical_and(vbits == t32, valid)
                eqrank = eqoff + plsc.cumsum(m_eq.astype(jnp.int32))
                take_eq = jnp.logical_and(m_eq, eqrank <= need_eq)
                m_sel = jnp.logical_or(m_lt, take_eq)
                pos = soff + plsc.cumsum(m_sel.astype(jnp.int32)) - 1
                pos = jnp.minimum(pos, 63)
                plsc.store_scatter(sel_idx, [pos], ci, mask=m_sel)
                soff = soff + plsc.all_reduce_population_count(m_sel)
                eqoff = eqoff + plsc.all_reduce_population_count(m_eq)
                return soff, eqoff

            lax.fori_loop(0, nv, emit_body,
                          (jnp.zeros((16,), jnp.int32),
                           jnp.zeros((16,), jnp.int32)))

            (ax, ay, az, axx, ayy, azz, axy, axz, ayz) = moms
            for s in range(2):
                gi = sel_idx[pl.ds(s * 16, 16)]
                gx = plsc.load_gather(xb, [gi]) - cxs
                gy = plsc.load_gather(yb, [gi]) - cys
                gz = plsc.load_gather(zb, [gi]) - czs
                base = r * (K * 3) + s * 48
                p0 = base + iota * 3
                plsc.store_scatter(pgblock, [p0], gx)
                plsc.store_scatter(pgblock, [p0 + 1], gy)
                plsc.store_scatter(pgblock, [p0 + 2], gz)
                ax = ax + gx
                ay = ay + gy
                az = az + gz
                axx = axx + gx * gx
                ayy = ayy + gy * gy
                azz = azz + gz * gz
                axy = axy + gx * gy
                axz = axz + gx * gz
                ayz = ayz + gy * gz
            return (ax, ay, az, axx, ayy, azz, axy, axz, ayz)

        moms = lax.fori_loop(0, G, row_body, tuple(zero16f for _ in range(9)))
        for k, acc in enumerate(moms):
            mombuf[pl.ds(k * 16, 16)] = acc
        pltpu.sync_copy(pgblock, pg_out.at[pl.ds(b * (G * K * 3), G * K * 3)])
        pltpu.sync_copy(mombuf, mom_out.at[b])

    pg_flat, mom = k3(d2, ptsT, cxyz, t0)
    return pg_flat.reshape(B * G * K, 3), mom


def _bn(x, g, b):
    mu = x.mean(axis=(0, 1), keepdims=True)
    var = x.var(axis=(0, 1), keepdims=True)
    return (x - mu) / jnp.sqrt(var + 1e-5) * g + b


def _fps(xyz, K):
    B_, N_, _ = xyz.shape
    dists = jnp.full((B_, N_), 1e10, dtype=xyz.dtype)
    farthest = jnp.zeros((B_,), dtype=jnp.int32)
    idx_list = []
    for _ in range(K):
        idx_list.append(farthest)
        centroid = jnp.take_along_axis(xyz, farthest[:, None, None], axis=1)
        d = jnp.sum((xyz - centroid) ** 2, axis=-1)
        dists = jnp.minimum(dists, d)
        farthest = jnp.argmax(dists, axis=1).astype(jnp.int32)
    idxs = jnp.stack(idx_list, axis=1)
    centers = jnp.take_along_axis(xyz, idxs[:, :, None], axis=1)
    return centers, idxs


def _morton(grid, perm):
    x = grid[..., perm[0]]
    y = grid[..., perm[1]]
    z = grid[..., perm[2]]
    code = jnp.zeros(x.shape, dtype=jnp.int32)
    for b in range(BITS):
        code = code | (((x >> b) & 1) << (3 * b + 2)) | (((y >> b) & 1) << (3 * b + 1)) | (((z >> b) & 1) << (3 * b))
    return code


def _serialize(pos, feats, perm):
    grid = jnp.floor(pos / GRID_SIZE).astype(jnp.int32)
    grid = grid - grid.min(axis=1, keepdims=True)
    grid = jnp.clip(grid, 0, (1 << BITS) - 1)
    code = _morton(grid, perm)
    order = jnp.argsort(code, axis=1)
    return [jnp.take_along_axis(f, order[:, :, None], axis=1) for f in feats]


def _final_pallas(x, pos_all):
    Bb, T, D = x.shape

    def body(x_ref, p_ref, o_ref):
        xs = x_ref[0] + p_ref[0]
        cls = xs[T - 1, :]
        mean = jnp.mean(xs, axis=0)
        o_ref[0, 0] = jnp.concatenate([cls, mean], axis=-1)

    out = pl.pallas_call(
        body,
        grid=(Bb,),
        in_specs=[
            pl.BlockSpec((1, T, D), lambda b: (b, 0, 0)),
            pl.BlockSpec((1, T, D), lambda b: (b, 0, 0)),
        ],
        out_specs=pl.BlockSpec((1, 1, 2 * D), lambda b: (b, 0, 0)),
        out_shape=jax.ShapeDtypeStruct((Bb, 1, 2 * D), x.dtype),
    )(x, pos_all)
    return out.reshape(Bb, 2 * D)


def kernel(pts, W1, b1, g1, be1, W2, b2, W3, b3, g2, be2, W4, b4, Wp1, bp1, Wp2, bp2, gamma1, beta1, gamma2, beta2, cls_token, cls_pos):
    centers, _ = _fps(pts, NUM_GROUP)

    ptsT = jnp.transpose(pts, (2, 0, 1))
    cxyz = jnp.transpose(centers, (2, 0, 1))
    d2, t0 = _k2(ptsT, cxyz)
    pg_rows, _mom = _k3_sc(d2, ptsT, cxyz, t0)
    pg = pg_rows.reshape(B * NUM_GROUP, GROUP_SIZE, 3)

    f = pg @ W1.T + b1
    f = jax.nn.relu(_bn(f, g1, be1))
    f = f @ W2.T + b2
    fg = f.max(axis=1, keepdims=True)
    f = jnp.concatenate([jnp.broadcast_to(fg, f.shape), f], axis=-1)
    f = f @ W3.T + b3
    f = jax.nn.relu(_bn(f, g2, be2))
    f = f @ W4.T + b4
    tokens = f.max(axis=1).reshape(B, NUM_GROUP, ENC_DIM)
    pos = jax.nn.gelu(centers @ Wp1.T + bp1) @ Wp2.T + bp2
    tok_f, pos_f = _serialize(centers, [tokens, pos], (0, 1, 2))
    tok_b, pos_b = _serialize(centers, [tokens, pos], (2, 1, 0))
    tok_f = tok_f * gamma1 + beta1
    tok_b = tok_b * gamma2 + beta2
    cls_t = jnp.broadcast_to(cls_token, (B, 1, TRANS_DIM))
    cls_p = jnp.broadcast_to(cls_pos, (B, 1, TRANS_DIM))
    pos_all = jnp.concatenate([pos_f, pos_b, cls_t], axis=1)
    x = jnp.concatenate([tok_f, tok_b, cls_p], axis=1)
    return _final_pallas(x, pos_all)


# speedup vs baseline: 5.0543x; 1.3537x over previous
import functools

import jax
import jax.numpy as jnp
from jax import lax
from jax.experimental import pallas as pl
from jax.experimental.pallas import tpu as pltpu
from jax.experimental.pallas import tpu_sc as plsc

TRANS_DIM = 384
ENC_DIM = 384
NUM_GROUP = 128
GROUP_SIZE = 32
GRID_SIZE = 0.02
BITS = 10

B = 32
N = 8192
NROWS = B * NUM_GROUP
CAND_CAP = 4096


def _k1_body(pts_ref, c_ref):
    x = pts_ref[0]
    y = pts_ref[1]
    z = pts_ref[2]
    iota_n = lax.broadcasted_iota(jnp.int32, (B, N), 1)
    iota_g = lax.broadcasted_iota(jnp.int32, (B, NUM_GROUP), 1)

    def body(k, st):
        dists, far, cxs, cys, czs = st
        oh = iota_n == far
        cx = jnp.sum(jnp.where(oh, x, 0.0), axis=1, keepdims=True)
        cy = jnp.sum(jnp.where(oh, y, 0.0), axis=1, keepdims=True)
        cz = jnp.sum(jnp.where(oh, z, 0.0), axis=1, keepdims=True)
        sel = iota_g == k
        cxs = jnp.where(sel, cx, cxs)
        cys = jnp.where(sel, cy, cys)
        czs = jnp.where(sel, cz, czs)
        dx = x - cx
        dy = y - cy
        dz = z - cz
        d = (dx * dx + dy * dy) + dz * dz
        dists = jnp.minimum(dists, d)
        m = jnp.max(dists, axis=1, keepdims=True)
        far = jnp.min(jnp.where(dists == m, iota_n, N), axis=1, keepdims=True)
        return dists, far, cxs, cys, czs

    init = (
        jnp.full((B, N), 1e10, jnp.float32),
        jnp.zeros((B, 1), jnp.int32),
        jnp.zeros((B, NUM_GROUP), jnp.float32),
        jnp.zeros((B, NUM_GROUP), jnp.float32),
        jnp.zeros((B, NUM_GROUP), jnp.float32),
    )
    _, _, cxs, cys, czs = lax.fori_loop(0, NUM_GROUP, body, init)
    c_ref[0] = cxs
    c_ref[1] = cys
    c_ref[2] = czs


def _k1(ptsT):
    return pl.pallas_call(
        _k1_body,
        in_specs=[pl.BlockSpec((3, B, N), lambda: (0, 0, 0))],
        out_specs=pl.BlockSpec((3, B, NUM_GROUP), lambda: (0, 0, 0)),
        out_shape=jax.ShapeDtypeStruct((3, B, NUM_GROUP), jnp.float32),
    )(ptsT)


def _k2_body(pts_ref, c_ref, d2_ref, t0_ref):
    pts_mat = pts_ref[:, 0, 0, :]
    c_mat = c_ref[:, 0, 0, :]
    x2 = pts_mat * pts_mat
    pp = (x2[0:1] + x2[1:2]) + x2[2:3]
    c2 = c_mat * c_mat
    cc = (c2[0:1] + c2[1:2]) + c2[2:3]
    prod = jax.lax.dot_general(
        c_mat, pts_mat, (((0,), (0,)), ((), ())),
        preferred_element_type=jnp.float32)
    d2 = (jnp.transpose(cc) + pp) - 2.0 * prod
    d2_ref[...] = d2
    m = jnp.min(d2.reshape(NUM_GROUP, 32, N // 32), axis=2)
    t0_ref[0, 0] = jnp.max(m, axis=1)


def _k2(ptsT, cxyz):
    d2, t0 = pl.pallas_call(
        _k2_body,
        grid=(B,),
        in_specs=[
            pl.BlockSpec((3, 1, 1, N), lambda b: (0, b, 0, 0)),
            pl.BlockSpec((3, 1, 1, NUM_GROUP), lambda b: (0, b, 0, 0)),
        ],
        out_specs=[
            pl.BlockSpec((NUM_GROUP, N), lambda b: (b, 0)),
            pl.BlockSpec((1, 1, NUM_GROUP), lambda b: (b, 0, 0)),
        ],
        out_shape=[
            jax.ShapeDtypeStruct((NROWS, N), jnp.float32),
            jax.ShapeDtypeStruct((B, 1, NUM_GROUP), jnp.float32),
        ],
    )(ptsT.reshape(3, B, 1, N), cxyz.reshape(3, B, 1, NUM_GROUP))
    return d2, t0.reshape(B, NUM_GROUP)


def _k3_sc(d2, ptsT, cxyz, t0):
    info = plsc.get_sparse_core_info()
    NC = info.num_cores

    mesh = plsc.VectorSubcoreMesh(core_axis_name="c", subcore_axis_name="s")
    G = NUM_GROUP
    K = GROUP_SIZE

    @functools.partial(
        pl.kernel,
        mesh=mesh,
        compiler_params=pltpu.CompilerParams(needs_layout_passes=False),
        out_type=[
            jax.ShapeDtypeStruct((B * G * K * 3,), jnp.float32),
            jax.ShapeDtypeStruct((B, 144), jnp.float32),
        ],
        scratch_types=[
            pltpu.VMEM((N,), jnp.float32),
            pltpu.VMEM((N,), jnp.float32),
            pltpu.VMEM((N,), jnp.float32),
            pltpu.VMEM((G,), jnp.float32),
            pltpu.VMEM((G,), jnp.float32),
            pltpu.VMEM((G,), jnp.float32),
            pltpu.VMEM((G,), jnp.float32),
            pltpu.VMEM((N,), jnp.float32),
            pltpu.VMEM((CAND_CAP,), jnp.float32),
            pltpu.VMEM((CAND_CAP,), jnp.int32),
            pltpu.VMEM((64,), jnp.int32),
            pltpu.VMEM((G * K * 3,), jnp.float32),
            pltpu.VMEM((144,), jnp.float32),
        ],
    )
    def k3(d2_hbm, ptsT_hbm, cxyz_hbm, t0_hbm, pg_out, mom_out,
           xb, yb, zb, cxb, cyb, czb, t0b, drow, cand_v, cand_i,
           sel_idx, pgblock, mombuf):
        b = lax.axis_index("s") * NC + lax.axis_index("c")

        pltpu.sync_copy(ptsT_hbm.at[0, b], xb)
        pltpu.sync_copy(ptsT_hbm.at[1, b], yb)
        pltpu.sync_copy(ptsT_hbm.at[2, b], zb)
        pltpu.sync_copy(cxyz_hbm.at[0, b], cxb)
        pltpu.sync_copy(cxyz_hbm.at[1, b], cyb)
        pltpu.sync_copy(cxyz_hbm.at[2, b], czb)
        pltpu.sync_copy(t0_hbm.at[b], t0b)

        iota = lax.iota(jnp.int32, 16)
        zero16f = jnp.zeros((16,), jnp.float32)

        def row_body(r, moms):
            pltpu.sync_copy(d2_hbm.at[b * G + r], drow)
            rsp = jnp.full((16,), r, jnp.int32)
            t0s = plsc.load_gather(t0b, [rsp])
            cxs = plsc.load_gather(cxb, [rsp])
            cys = plsc.load_gather(cyb, [rsp])
            czs = plsc.load_gather(czb, [rsp])

            def scan_body(j, off):
                v = drow[pl.ds(j * 16, 16)]
                m = v <= t0s
                mi = m.astype(jnp.int32)
                pos = off + plsc.cumsum(mi) - 1
                pos = jnp.minimum(pos, CAND_CAP - 1)
                idxv = iota + j * 16
                plsc.store_scatter(cand_v, [pos], v, mask=m)
                plsc.store_scatter(cand_i, [pos], idxv, mask=m)
                return off + plsc.all_reduce_population_count(m)

            cnt_sp = lax.fori_loop(0, N // 16, scan_body,
                                   jnp.zeros((16,), jnp.int32))
            cnt = jnp.max(cnt_sp)
            nv = (cnt + 15) // 16

            def count_le(mid_sp):
                def cbody(c, acc):
                    vbits = plsc.bitcast(cand_v[pl.ds(c * 16, 16)], jnp.int32)
                    valid = (iota + c * 16) < cnt_sp
                    le = jnp.logical_and(vbits <= mid_sp, valid)
                    return acc + plsc.all_reduce_population_count(le)
                return lax.fori_loop(0, nv, cbody, jnp.zeros((16,), jnp.int32))

            def bis_body(_, st):
                lo, hi, cnt_lo = st
                mid = jnp.right_shift(lo + hi, 1)
                c = count_le(mid)
                ge = c >= 32
                hi = jnp.where(ge, mid, hi)
                new_lo = jnp.where(ge, lo, mid)
                cnt_lo = jnp.where(ge, cnt_lo, c)
                return new_lo, hi, cnt_lo

            lo0 = jnp.full((16,), jnp.int32(-2147483647), jnp.int32)
            hi0 = jnp.full((16,), jnp.int32(0x7F800000), jnp.int32)
            lo, hi, cnt_lo = lax.fori_loop(
                0, 32, bis_body, (lo0, hi0, jnp.zeros((16,), jnp.int32)))
            t32 = hi
            need_eq = 32 - cnt_lo

            def emit_body(c, st):
                soff, eqoff = st
                vbits = plsc.bitcast(cand_v[pl.ds(c * 16, 16)], jnp.int32)
                ci = cand_i[pl.ds(c * 16, 16)]
                valid = (iota + c * 16) < cnt_sp
                m_lt = jnp.logical_and(vbits < t32, valid)
                m_eq = jnp.logical_and(vbits == t32, valid)
                eqrank = eqoff + plsc.cumsum(m_eq.astype(jnp.int32))
                take_eq = jnp.logical_and(m_eq, eqrank <= need_eq)
                m_sel = jnp.logical_or(m_lt, take_eq)
                pos = soff + plsc.cumsum(m_sel.astype(jnp.int32)) - 1
                pos = jnp.minimum(pos, 63)
                plsc.store_scatter(sel_idx, [pos], ci, mask=m_sel)
                soff = soff + plsc.all_reduce_population_count(m_sel)
                eqoff = eqoff + plsc.all_reduce_population_count(m_eq)
                return soff, eqoff

            lax.fori_loop(0, nv, emit_body,
                          (jnp.zeros((16,), jnp.int32),
                           jnp.zeros((16,), jnp.int32)))

            (ax, ay, az, axx, ayy, azz, axy, axz, ayz) = moms
            for s in range(2):
                gi = sel_idx[pl.ds(s * 16, 16)]
                gx = plsc.load_gather(xb, [gi]) - cxs
                gy = plsc.load_gather(yb, [gi]) - cys
                gz = plsc.load_gather(zb, [gi]) - czs
                base = r * (K * 3) + s * 48
                p0 = base + iota * 3
                plsc.store_scatter(pgblock, [p0], gx)
                plsc.store_scatter(pgblock, [p0 + 1], gy)
                plsc.store_scatter(pgblock, [p0 + 2], gz)
                ax = ax + gx
                ay = ay + gy
                az = az + gz
                axx = axx + gx * gx
                ayy = ayy + gy * gy
                azz = azz + gz * gz
                axy = axy + gx * gy
                axz = axz + gx * gz
                ayz = ayz + gy * gz
            return (ax, ay, az, axx, ayy, azz, axy, axz, ayz)

        moms = lax.fori_loop(0, G, row_body, tuple(zero16f for _ in range(9)))
        for k, acc in enumerate(moms):
            mombuf[pl.ds(k * 16, 16)] = acc
        pltpu.sync_copy(pgblock, pg_out.at[pl.ds(b * (G * K * 3), G * K * 3)])
        pltpu.sync_copy(mombuf, mom_out.at[b])

    pg_flat, mom = k3(d2, ptsT, cxyz, t0)
    return pg_flat.reshape(B * G * K, 3), mom


def _bn(x, g, b):
    mu = x.mean(axis=(0, 1), keepdims=True)
    var = x.var(axis=(0, 1), keepdims=True)
    return (x - mu) / jnp.sqrt(var + 1e-5) * g + b


def _fps(xyz, K):
    B_, N_, _ = xyz.shape
    dists = jnp.full((B_, N_), 1e10, dtype=xyz.dtype)
    farthest = jnp.zeros((B_,), dtype=jnp.int32)
    idx_list = []
    for _ in range(K):
        idx_list.append(farthest)
        centroid = jnp.take_along_axis(xyz, farthest[:, None, None], axis=1)
        d = jnp.sum((xyz - centroid) ** 2, axis=-1)
        dists = jnp.minimum(dists, d)
        farthest = jnp.argmax(dists, axis=1).astype(jnp.int32)
    idxs = jnp.stack(idx_list, axis=1)
    centers = jnp.take_along_axis(xyz, idxs[:, :, None], axis=1)
    return centers, idxs


def _morton(grid, perm):
    x = grid[..., perm[0]]
    y = grid[..., perm[1]]
    z = grid[..., perm[2]]
    code = jnp.zeros(x.shape, dtype=jnp.int32)
    for b in range(BITS):
        code = code | (((x >> b) & 1) << (3 * b + 2)) | (((y >> b) & 1) << (3 * b + 1)) | (((z >> b) & 1) << (3 * b))
    return code


def _serialize(pos, feats, perm):
    grid = jnp.floor(pos / GRID_SIZE).astype(jnp.int32)
    grid = grid - grid.min(axis=1, keepdims=True)
    grid = jnp.clip(grid, 0, (1 << BITS) - 1)
    code = _morton(grid, perm)
    order = jnp.argsort(code, axis=1)
    return [jnp.take_along_axis(f, order[:, :, None], axis=1) for f in feats]


def _final_pallas(x, pos_all):
    Bb, T, D = x.shape

    def body(x_ref, p_ref, o_ref):
        xs = x_ref[0] + p_ref[0]
        cls = xs[T - 1, :]
        mean = jnp.mean(xs, axis=0)
        o_ref[0, 0] = jnp.concatenate([cls, mean], axis=-1)

    out = pl.pallas_call(
        body,
        grid=(Bb,),
        in_specs=[
            pl.BlockSpec((1, T, D), lambda b: (b, 0, 0)),
            pl.BlockSpec((1, T, D), lambda b: (b, 0, 0)),
        ],
        out_specs=pl.BlockSpec((1, 1, 2 * D), lambda b: (b, 0, 0)),
        out_shape=jax.ShapeDtypeStruct((Bb, 1, 2 * D), x.dtype),
    )(x, pos_all)
    return out.reshape(Bb, 2 * D)


def kernel(pts, W1, b1, g1, be1, W2, b2, W3, b3, g2, be2, W4, b4, Wp1, bp1, Wp2, bp2, gamma1, beta1, gamma2, beta2, cls_token, cls_pos):
    ptsT = jnp.transpose(pts, (2, 0, 1))
    cxyz = _k1(ptsT)
    centers = jnp.transpose(cxyz, (1, 2, 0))
    d2, t0 = _k2(ptsT, cxyz)
    pg_rows, _mom = _k3_sc(d2, ptsT, cxyz, t0)
    pg = pg_rows.reshape(B * NUM_GROUP, GROUP_SIZE, 3)

    f = pg @ W1.T + b1
    f = jax.nn.relu(_bn(f, g1, be1))
    f = f @ W2.T + b2
    fg = f.max(axis=1, keepdims=True)
    f = jnp.concatenate([jnp.broadcast_to(fg, f.shape), f], axis=-1)
    f = f @ W3.T + b3
    f = jax.nn.relu(_bn(f, g2, be2))
    f = f @ W4.T + b4
    tokens = f.max(axis=1).reshape(B, NUM_GROUP, ENC_DIM)
    pos = jax.nn.gelu(centers @ Wp1.T + bp1) @ Wp2.T + bp2
    tok_f, pos_f = _serialize(centers, [tokens, pos], (0, 1, 2))
    tok_b, pos_b = _serialize(centers, [tokens, pos], (2, 1, 0))
    tok_f = tok_f * gamma1 + beta1
    tok_b = tok_b * gamma2 + beta2
    cls_t = jnp.broadcast_to(cls_token, (B, 1, TRANS_DIM))
    cls_p = jnp.broadcast_to(cls_pos, (B, 1, TRANS_DIM))
    pos_all = jnp.concatenate([pos_f, pos_b, cls_t], axis=1)
    x = jnp.concatenate([tok_f, tok_b, cls_p], axis=1)
    return _final_pallas(x, pos_all)


# speedup vs baseline: 5.2321x; 1.0352x over previous
import functools

import jax
import jax.numpy as jnp
from jax import lax
from jax.experimental import pallas as pl
from jax.experimental.pallas import tpu as pltpu
from jax.experimental.pallas import tpu_sc as plsc

TRANS_DIM = 384
ENC_DIM = 384
NUM_GROUP = 128
GROUP_SIZE = 32
GRID_SIZE = 0.02
BITS = 10

B = 32
N = 8192
NROWS = B * NUM_GROUP
CAND_CAP = 4096


def _k1_body(pts_ref, c_ref):
    x = pts_ref[0]
    y = pts_ref[1]
    z = pts_ref[2]
    iota_n = lax.broadcasted_iota(jnp.int32, (B, N), 1)
    iota_g = lax.broadcasted_iota(jnp.int32, (B, NUM_GROUP), 1)

    def body(k, st):
        dists, far, cxs, cys, czs = st
        oh = iota_n == far
        cx = jnp.sum(jnp.where(oh, x, 0.0), axis=1, keepdims=True)
        cy = jnp.sum(jnp.where(oh, y, 0.0), axis=1, keepdims=True)
        cz = jnp.sum(jnp.where(oh, z, 0.0), axis=1, keepdims=True)
        sel = iota_g == k
        cxs = jnp.where(sel, cx, cxs)
        cys = jnp.where(sel, cy, cys)
        czs = jnp.where(sel, cz, czs)
        dx = x - cx
        dy = y - cy
        dz = z - cz
        d = (dx * dx + dy * dy) + dz * dz
        dists = jnp.minimum(dists, d)
        m = jnp.max(dists, axis=1, keepdims=True)
        far = jnp.min(jnp.where(dists == m, iota_n, N), axis=1, keepdims=True)
        return dists, far, cxs, cys, czs

    init = (
        jnp.full((B, N), 1e10, jnp.float32),
        jnp.zeros((B, 1), jnp.int32),
        jnp.zeros((B, NUM_GROUP), jnp.float32),
        jnp.zeros((B, NUM_GROUP), jnp.float32),
        jnp.zeros((B, NUM_GROUP), jnp.float32),
    )
    _, _, cxs, cys, czs = lax.fori_loop(0, NUM_GROUP, body, init)
    c_ref[0] = cxs
    c_ref[1] = cys
    c_ref[2] = czs


def _k1(ptsT):
    return pl.pallas_call(
        _k1_body,
        in_specs=[pl.BlockSpec((3, B, N), lambda: (0, 0, 0))],
        out_specs=pl.BlockSpec((3, B, NUM_GROUP), lambda: (0, 0, 0)),
        out_shape=jax.ShapeDtypeStruct((3, B, NUM_GROUP), jnp.float32),
    )(ptsT)


def _k2_body(pts_ref, c_ref, d2_ref, t0_ref):
    pts_mat = pts_ref[:, 0, 0, :]
    c_mat = c_ref[:, 0, 0, :]
    x2 = pts_mat * pts_mat
    pp = (x2[0:1] + x2[1:2]) + x2[2:3]
    c2 = c_mat * c_mat
    cc = (c2[0:1] + c2[1:2]) + c2[2:3]
    prod = jax.lax.dot_general(
        c_mat, pts_mat, (((0,), (0,)), ((), ())),
        preferred_element_type=jnp.float32)
    d2 = (jnp.transpose(cc) + pp) - 2.0 * prod
    d2_ref[...] = d2
    m = jnp.min(d2.reshape(NUM_GROUP, 32, N // 32), axis=2)
    t0_ref[0, 0] = jnp.max(m, axis=1)


def _k2(ptsT, cxyz):
    d2, t0 = pl.pallas_call(
        _k2_body,
        grid=(B,),
        in_specs=[
            pl.BlockSpec((3, 1, 1, N), lambda b: (0, b, 0, 0)),
            pl.BlockSpec((3, 1, 1, NUM_GROUP), lambda b: (0, b, 0, 0)),
        ],
        out_specs=[
            pl.BlockSpec((NUM_GROUP, N), lambda b: (b, 0)),
            pl.BlockSpec((1, 1, NUM_GROUP), lambda b: (b, 0, 0)),
        ],
        out_shape=[
            jax.ShapeDtypeStruct((NROWS, N), jnp.float32),
            jax.ShapeDtypeStruct((B, 1, NUM_GROUP), jnp.float32),
        ],
    )(ptsT.reshape(3, B, 1, N), cxyz.reshape(3, B, 1, NUM_GROUP))
    return d2, t0.reshape(B, NUM_GROUP)


def _k3_sc(d2, ptsT, cxyz, t0):
    info = plsc.get_sparse_core_info()
    NC = info.num_cores

    mesh = plsc.VectorSubcoreMesh(core_axis_name="c", subcore_axis_name="s")
    G = NUM_GROUP
    K = GROUP_SIZE

    @functools.partial(
        pl.kernel,
        mesh=mesh,
        compiler_params=pltpu.CompilerParams(needs_layout_passes=False),
        out_type=[
            jax.ShapeDtypeStruct((B * G * K * 3,), jnp.float32),
            jax.ShapeDtypeStruct((B, 144), jnp.float32),
        ],
        scratch_types=[
            pltpu.VMEM((N,), jnp.float32),
            pltpu.VMEM((N,), jnp.float32),
            pltpu.VMEM((N,), jnp.float32),
            pltpu.VMEM((G,), jnp.float32),
            pltpu.VMEM((G,), jnp.float32),
            pltpu.VMEM((G,), jnp.float32),
            pltpu.VMEM((G,), jnp.float32),
            pltpu.VMEM((N,), jnp.float32),
            pltpu.VMEM((CAND_CAP,), jnp.float32),
            pltpu.VMEM((CAND_CAP,), jnp.int32),
            pltpu.VMEM((64,), jnp.int32),
            pltpu.VMEM((G * K * 3,), jnp.float32),
            pltpu.VMEM((144,), jnp.float32),
        ],
    )
    def k3(d2_hbm, ptsT_hbm, cxyz_hbm, t0_hbm, pg_out, mom_out,
           xb, yb, zb, cxb, cyb, czb, t0b, drow, cand_v, cand_i,
           sel_idx, pgblock, mombuf):
        b = lax.axis_index("s") * NC + lax.axis_index("c")

        pltpu.sync_copy(ptsT_hbm.at[0, b], xb)
        pltpu.sync_copy(ptsT_hbm.at[1, b], yb)
        pltpu.sync_copy(ptsT_hbm.at[2, b], zb)
        pltpu.sync_copy(cxyz_hbm.at[0, b], cxb)
        pltpu.sync_copy(cxyz_hbm.at[1, b], cyb)
        pltpu.sync_copy(cxyz_hbm.at[2, b], czb)
        pltpu.sync_copy(t0_hbm.at[b], t0b)

        iota = lax.iota(jnp.int32, 16)
        zero16f = jnp.zeros((16,), jnp.float32)

        def row_body(r, moms):
            pltpu.sync_copy(d2_hbm.at[b * G + r], drow)
            rsp = jnp.full((16,), r, jnp.int32)
            t0s = plsc.load_gather(t0b, [rsp])
            cxs = plsc.load_gather(cxb, [rsp])
            cys = plsc.load_gather(cyb, [rsp])
            czs = plsc.load_gather(czb, [rsp])

            def scan_body(j, off):
                v = drow[pl.ds(j * 16, 16)]
                m = v <= t0s
                mi = m.astype(jnp.int32)
                pos = off + plsc.cumsum(mi) - 1
                pos = jnp.minimum(pos, CAND_CAP - 1)
                idxv = iota + j * 16
                plsc.store_scatter(cand_v, [pos], v, mask=m)
                plsc.store_scatter(cand_i, [pos], idxv, mask=m)
                return off + plsc.all_reduce_population_count(m)

            cnt_sp = lax.fori_loop(0, N // 16, scan_body,
                                   jnp.zeros((16,), jnp.int32))
            cnt = jnp.max(cnt_sp)
            nv = (cnt + 15) // 16

            def count_le(mid_sp):
                def cbody(c, acc):
                    vbits = plsc.bitcast(cand_v[pl.ds(c * 16, 16)], jnp.int32)
                    valid = (iota + c * 16) < cnt_sp
                    le = jnp.logical_and(vbits <= mid_sp, valid)
                    return acc + plsc.all_reduce_population_count(le)
                return lax.fori_loop(0, nv, cbody, jnp.zeros((16,), jnp.int32))

            def bis_body(_, st):
                lo, hi, cnt_lo = st
                mid = jnp.right_shift(lo + hi, 1)
                c = count_le(mid)
                ge = c >= 32
                hi = jnp.where(ge, mid, hi)
                new_lo = jnp.where(ge, lo, mid)
                cnt_lo = jnp.where(ge, cnt_lo, c)
                return new_lo, hi, cnt_lo

            lo0 = jnp.full((16,), jnp.int32(-2147483647), jnp.int32)
            hi0 = jnp.full((16,), jnp.int32(0x7F800000), jnp.int32)
            lo, hi, cnt_lo = lax.fori_loop(
                0, 32, bis_body, (lo0, hi0, jnp.zeros((16,), jnp.int32)))
            t32 = hi
            need_eq = 32 - cnt_lo

            def emit_body(c, st):
                soff, eqoff = st
                vbits = plsc.bitcast(cand_v[pl.ds(c * 16, 16)], jnp.int32)
                ci = cand_i[pl.ds(c * 16, 16)]
                valid = (iota + c * 16) < cnt_sp
                m_lt = jnp.logical_and(vbits < t32, valid)
                m_eq = jnp.logical_and(vbits == t32, valid)
                eqrank = eqoff + plsc.cumsum(m_eq.astype(jnp.int32))
                take_eq = jnp.logical_and(m_eq, eqrank <= need_eq)
                m_sel = jnp.logical_or(m_lt, take_eq)
                pos = soff + plsc.cumsum(m_sel.astype(jnp.int32)) - 1
                pos = jnp.minimum(pos, 63)
                plsc.store_scatter(sel_idx, [pos], ci, mask=m_sel)
                soff = soff + plsc.all_reduce_population_count(m_sel)
                eqoff = eqoff + plsc.all_reduce_population_count(m_eq)
                return soff, eqoff

            lax.fori_loop(0, nv, emit_body,
                          (jnp.zeros((16,), jnp.int32),
                           jnp.zeros((16,), jnp.int32)))

            (ax, ay, az, axx, ayy, azz, axy, axz, ayz) = moms
            for s in range(2):
                gi = sel_idx[pl.ds(s * 16, 16)]
                gx = plsc.load_gather(xb, [gi]) - cxs
                gy = plsc.load_gather(yb, [gi]) - cys
                gz = plsc.load_gather(zb, [gi]) - czs
                base = r * (K * 3) + s * 48
                p0 = base + iota * 3
                plsc.store_scatter(pgblock, [p0], gx)
                plsc.store_scatter(pgblock, [p0 + 1], gy)
                plsc.store_scatter(pgblock, [p0 + 2], gz)
                ax = ax + gx
                ay = ay + gy
                az = az + gz
                axx = axx + gx * gx
                ayy = ayy + gy * gy
                azz = azz + gz * gz
                axy = axy + gx * gy
                axz = axz + gx * gz
                ayz = ayz + gy * gz
            return (ax, ay, az, axx, ayy, azz, axy, axz, ayz)

        moms = lax.fori_loop(0, G, row_body, tuple(zero16f for _ in range(9)))
        for k, acc in enumerate(moms):
            mombuf[pl.ds(k * 16, 16)] = acc
        pltpu.sync_copy(pgblock, pg_out.at[pl.ds(b * (G * K * 3), G * K * 3)])
        pltpu.sync_copy(mombuf, mom_out.at[b])

    pg_flat, mom = k3(d2, ptsT, cxyz, t0)
    return pg_flat.reshape(B * G * K, 3), mom


_R = 1024
_NSTEPS = (B * NUM_GROUP * GROUP_SIZE) // _R
_NPTS = float(B * NUM_GROUP * GROUP_SIZE)


def _k4b_body(pg_ref, mom_ref, W1_ref, b1_ref, g1_ref, be1_ref,
              W2_ref, b2_ref, W3_ref, b3_ref, f3_ref, stats_ref, acc_ref):
    i = pl.program_id(0)

    @pl.when(i == 0)
    def _():
        acc_ref[...] = jnp.zeros_like(acc_ref)

    momr = mom_ref[...]
    s = [jnp.sum(momr[:, k * 16:(k + 1) * 16]) for k in range(9)]
    mx, my, mz = s[0] / _NPTS, s[1] / _NPTS, s[2] / _NPTS
    sxx, syy, szz = s[3] / _NPTS, s[4] / _NPTS, s[5] / _NPTS
    sxy, sxz, syz = s[6] / _NPTS, s[7] / _NPTS, s[8] / _NPTS

    w = W1_ref[...]
    wx, wy, wz = w[:, 0], w[:, 1], w[:, 2]
    lin = wx * mx + wy * my + wz * mz
    quad = (wx * wx * sxx + wy * wy * syy + wz * wz * szz
            + 2.0 * (wx * wy * sxy + wx * wz * sxz + wy * wz * syz))
    var1 = quad - lin * lin
    mu1 = lin + b1_ref[...]
    s1 = g1_ref[...] / jnp.sqrt(var1 + 1e-5)
    t1 = be1_ref[...] - mu1 * s1

    pg = pg_ref[...]
    f1 = lax.dot_general(pg, w, (((1,), (1,)), ((), ())),
                         preferred_element_type=jnp.float32) + b1_ref[...]
    f1 = jax.nn.relu(f1 * s1 + t1)
    f2 = lax.dot_general(f1, W2_ref[...], (((1,), (1,)), ((), ())),
                         preferred_element_type=jnp.float32) + b2_ref[...]
    g = f2.reshape(_R // GROUP_SIZE, GROUP_SIZE, 256)
    fg = jnp.max(g, axis=1, keepdims=True)
    h = jnp.concatenate([jnp.broadcast_to(fg, g.shape), g], axis=-1)
    h = h.reshape(_R, 512)
    f3 = lax.dot_general(h, W3_ref[...], (((1,), (1,)), ((), ())),
                         preferred_element_type=jnp.float32) + b3_ref[...]
    f3_ref[...] = f3
    acc_ref[0, :] += jnp.sum(f3, axis=0)
    acc_ref[1, :] += jnp.sum(f3 * f3, axis=0)
    stats_ref[...] = acc_ref[...]


def _k4c_body(f3_ref, stats_ref, g2_ref, be2_ref, W4_ref, b4_ref, tok_ref):
    mu = stats_ref[0, :] / _NPTS
    var = stats_ref[1, :] / _NPTS - mu * mu
    s2 = g2_ref[...] / jnp.sqrt(var + 1e-5)
    t2 = be2_ref[...] - mu * s2
    r = jax.nn.relu(f3_ref[...] * s2 + t2)
    f4 = lax.dot_general(r, W4_ref[...], (((1,), (1,)), ((), ())),
                         preferred_element_type=jnp.float32) + b4_ref[...]
    tok_ref[...] = jnp.max(f4.reshape(_R // GROUP_SIZE, GROUP_SIZE, ENC_DIM),
                           axis=1)


def _k4(pg_rows, mom, W1, b1, g1, be1, W2, b2, W3, b3, g2, be2, W4, b4):
    full = lambda s: pl.BlockSpec(s, lambda i: tuple(0 for _ in s))
    f3, _stats = pl.pallas_call(
        _k4b_body,
        grid=(_NSTEPS,),
        in_specs=[
            pl.BlockSpec((_R, 3), lambda i: (i, 0)),
            full((B, 144)), full((128, 3)), full((128,)), full((128,)),
            full((128,)), full((256, 128)), full((256,)),
            full((512, 512)), full((512,)),
        ],
        out_specs=[
            pl.BlockSpec((_R, 512), lambda i: (i, 0)),
            pl.BlockSpec((2, 512), lambda i: (0, 0)),
        ],
        out_shape=[
            jax.ShapeDtypeStruct((B * NUM_GROUP * GROUP_SIZE, 512), jnp.float32),
            jax.ShapeDtypeStruct((2, 512), jnp.float32),
        ],
        scratch_shapes=[pltpu.VMEM((2, 512), jnp.float32)],
    )(pg_rows, mom, W1, b1, g1, be1, W2, b2, W3, b3)

    tokens = pl.pallas_call(
        _k4c_body,
        grid=(_NSTEPS,),
        in_specs=[
            pl.BlockSpec((_R, 512), lambda i: (i, 0)),
            full((2, 512)), full((512,)), full((512,)),
            full((ENC_DIM, 512)), full((ENC_DIM,)),
        ],
        out_specs=pl.BlockSpec((_R // GROUP_SIZE, ENC_DIM), lambda i: (i, 0)),
        out_shape=jax.ShapeDtypeStruct((B * NUM_GROUP, ENC_DIM), jnp.float32),
    )(f3, _stats, g2, be2, W4, b4)
    return tokens


def _bn(x, g, b):
    mu = x.mean(axis=(0, 1), keepdims=True)
    var = x.var(axis=(0, 1), keepdims=True)
    return (x - mu) / jnp.sqrt(var + 1e-5) * g + b


def _fps(xyz, K):
    B_, N_, _ = xyz.shape
    dists = jnp.full((B_, N_), 1e10, dtype=xyz.dtype)
    farthest = jnp.zeros((B_,), dtype=jnp.int32)
    idx_list = []
    for _ in range(K):
        idx_list.append(farthest)
        centroid = jnp.take_along_axis(xyz, farthest[:, None, None], axis=1)
        d = jnp.sum((xyz - centroid) ** 2, axis=-1)
        dists = jnp.minimum(dists, d)
        farthest = jnp.argmax(dists, axis=1).astype(jnp.int32)
    idxs = jnp.stack(idx_list, axis=1)
    centers = jnp.take_along_axis(xyz, idxs[:, :, None], axis=1)
    return centers, idxs


def _morton(grid, perm):
    x = grid[..., perm[0]]
    y = grid[..., perm[1]]
    z = grid[..., perm[2]]
    code = jnp.zeros(x.shape, dtype=jnp.int32)
    for b in range(BITS):
        code = code | (((x >> b) & 1) << (3 * b + 2)) | (((y >> b) & 1) << (3 * b + 1)) | (((z >> b) & 1) << (3 * b))
    return code


def _serialize(pos, feats, perm):
    grid = jnp.floor(pos / GRID_SIZE).astype(jnp.int32)
    grid = grid - grid.min(axis=1, keepdims=True)
    grid = jnp.clip(grid, 0, (1 << BITS) - 1)
    code = _morton(grid, perm)
    order = jnp.argsort(code, axis=1)
    return [jnp.take_along_axis(f, order[:, :, None], axis=1) for f in feats]


def _final_pallas(x, pos_all):
    Bb, T, D = x.shape

    def body(x_ref, p_ref, o_ref):
        xs = x_ref[0] + p_ref[0]
        cls = xs[T - 1, :]
        mean = jnp.mean(xs, axis=0)
        o_ref[0, 0] = jnp.concatenate([cls, mean], axis=-1)

    out = pl.pallas_call(
        body,
        grid=(Bb,),
        in_specs=[
            pl.BlockSpec((1, T, D), lambda b: (b, 0, 0)),
            pl.BlockSpec((1, T, D), lambda b: (b, 0, 0)),
        ],
        out_specs=pl.BlockSpec((1, 1, 2 * D), lambda b: (b, 0, 0)),
        out_shape=jax.ShapeDtypeStruct((Bb, 1, 2 * D), x.dtype),
    )(x, pos_all)
    return out.reshape(Bb, 2 * D)


def kernel(pts, W1, b1, g1, be1, W2, b2, W3, b3, g2, be2, W4, b4, Wp1, bp1, Wp2, bp2, gamma1, beta1, gamma2, beta2, cls_token, cls_pos):
    ptsT = jnp.transpose(pts, (2, 0, 1))
    cxyz = _k1(ptsT)
    centers = jnp.transpose(cxyz, (1, 2, 0))
    d2, t0 = _k2(ptsT, cxyz)
    pg_rows, mom = _k3_sc(d2, ptsT, cxyz, t0)
    tokens = _k4(pg_rows, mom, W1, b1, g1, be1, W2, b2, W3, b3,
                 g2, be2, W4, b4).reshape(B, NUM_GROUP, ENC_DIM)
    pos = jax.nn.gelu(centers @ Wp1.T + bp1) @ Wp2.T + bp2
    tok_f, pos_f = _serialize(centers, [tokens, pos], (0, 1, 2))
    tok_b, pos_b = _serialize(centers, [tokens, pos], (2, 1, 0))
    tok_f = tok_f * gamma1 + beta1
    tok_b = tok_b * gamma2 + beta2
    cls_t = jnp.broadcast_to(cls_token, (B, 1, TRANS_DIM))
    cls_p = jnp.broadcast_to(cls_pos, (B, 1, TRANS_DIM))
    pos_all = jnp.concatenate([pos_f, pos_b, cls_t], axis=1)
    x = jnp.concatenate([tok_f, tok_b, cls_p], axis=1)
    return _final_pallas(x, pos_all)


# speedup vs baseline: 5.7155x; 1.0924x over previous
import functools

import jax
import jax.numpy as jnp
from jax import lax
from jax.experimental import pallas as pl
from jax.experimental.pallas import tpu as pltpu
from jax.experimental.pallas import tpu_sc as plsc

TRANS_DIM = 384
ENC_DIM = 384
NUM_GROUP = 128
GROUP_SIZE = 32
GRID_SIZE = 0.02
BITS = 10

B = 32
N = 8192
NROWS = B * NUM_GROUP
CAND_CAP = 4096


def _k1_body(pts_ref, c_ref):
    x = pts_ref[0]
    y = pts_ref[1]
    z = pts_ref[2]
    iota_n = lax.broadcasted_iota(jnp.int32, (B, N), 1)
    iota_g = lax.broadcasted_iota(jnp.int32, (B, NUM_GROUP), 1)

    def body(k, st):
        dists, far, cxs, cys, czs = st
        oh = iota_n == far
        cx = jnp.sum(jnp.where(oh, x, 0.0), axis=1, keepdims=True)
        cy = jnp.sum(jnp.where(oh, y, 0.0), axis=1, keepdims=True)
        cz = jnp.sum(jnp.where(oh, z, 0.0), axis=1, keepdims=True)
        sel = iota_g == k
        cxs = jnp.where(sel, cx, cxs)
        cys = jnp.where(sel, cy, cys)
        czs = jnp.where(sel, cz, czs)
        dx = x - cx
        dy = y - cy
        dz = z - cz
        d = (dx * dx + dy * dy) + dz * dz
        dists = jnp.minimum(dists, d)
        m = jnp.max(dists, axis=1, keepdims=True)
        far = jnp.min(jnp.where(dists == m, iota_n, N), axis=1, keepdims=True)
        return dists, far, cxs, cys, czs

    init = (
        jnp.full((B, N), 1e10, jnp.float32),
        jnp.zeros((B, 1), jnp.int32),
        jnp.zeros((B, NUM_GROUP), jnp.float32),
        jnp.zeros((B, NUM_GROUP), jnp.float32),
        jnp.zeros((B, NUM_GROUP), jnp.float32),
    )
    _, _, cxs, cys, czs = lax.fori_loop(0, NUM_GROUP, body, init)
    c_ref[0] = cxs
    c_ref[1] = cys
    c_ref[2] = czs


def _k1(ptsT):
    return pl.pallas_call(
        _k1_body,
        in_specs=[pl.BlockSpec((3, B, N), lambda: (0, 0, 0))],
        out_specs=pl.BlockSpec((3, B, NUM_GROUP), lambda: (0, 0, 0)),
        out_shape=jax.ShapeDtypeStruct((3, B, NUM_GROUP), jnp.float32),
    )(ptsT)


def _k2_body(pts_ref, c_ref, d2_ref, t0_ref):
    pts_mat = pts_ref[:, 0, 0, :]
    c_mat = c_ref[:, 0, 0, :]
    x2 = pts_mat * pts_mat
    pp = (x2[0:1] + x2[1:2]) + x2[2:3]
    c2 = c_mat * c_mat
    cc = (c2[0:1] + c2[1:2]) + c2[2:3]
    prod = jax.lax.dot_general(
        c_mat, pts_mat, (((0,), (0,)), ((), ())),
        preferred_element_type=jnp.float32)
    d2 = (jnp.transpose(cc) + pp) - 2.0 * prod
    d2_ref[...] = d2
    m = jnp.min(d2.reshape(NUM_GROUP, 32, N // 32), axis=2)
    t0_ref[0, 0] = jnp.max(m, axis=1)


def _k2(ptsT, cxyz):
    d2, t0 = pl.pallas_call(
        _k2_body,
        grid=(B,),
        in_specs=[
            pl.BlockSpec((3, 1, 1, N), lambda b: (0, b, 0, 0)),
            pl.BlockSpec((3, 1, 1, NUM_GROUP), lambda b: (0, b, 0, 0)),
        ],
        out_specs=[
            pl.BlockSpec((NUM_GROUP, N), lambda b: (b, 0)),
            pl.BlockSpec((1, 1, NUM_GROUP), lambda b: (b, 0, 0)),
        ],
        out_shape=[
            jax.ShapeDtypeStruct((NROWS, N), jnp.float32),
            jax.ShapeDtypeStruct((B, 1, NUM_GROUP), jnp.float32),
        ],
    )(ptsT.reshape(3, B, 1, N), cxyz.reshape(3, B, 1, NUM_GROUP))
    return d2, t0.reshape(B, NUM_GROUP)


def _k3_sc(d2, ptsT, cxyz, t0):
    info = plsc.get_sparse_core_info()
    NC = info.num_cores

    mesh = plsc.VectorSubcoreMesh(core_axis_name="c", subcore_axis_name="s")
    G = NUM_GROUP
    K = GROUP_SIZE

    @functools.partial(
        pl.kernel,
        mesh=mesh,
        compiler_params=pltpu.CompilerParams(needs_layout_passes=False),
        out_type=[
            jax.ShapeDtypeStruct((B * G * K * 3,), jnp.float32),
            jax.ShapeDtypeStruct((B, 144), jnp.float32),
        ],
        scratch_types=[
            pltpu.VMEM((N,), jnp.float32),
            pltpu.VMEM((N,), jnp.float32),
            pltpu.VMEM((N,), jnp.float32),
            pltpu.VMEM((G,), jnp.float32),
            pltpu.VMEM((G,), jnp.float32),
            pltpu.VMEM((G,), jnp.float32),
            pltpu.VMEM((G,), jnp.float32),
            pltpu.VMEM((N,), jnp.float32),
            pltpu.VMEM((N,), jnp.float32),
            pltpu.SemaphoreType.DMA,
            pltpu.SemaphoreType.DMA,
            pltpu.VMEM((CAND_CAP + 64,), jnp.float32),
            pltpu.VMEM((CAND_CAP + 64,), jnp.int32),
            pltpu.VMEM((64,), jnp.int32),
            pltpu.VMEM((G * K * 3,), jnp.float32),
            pltpu.VMEM((144,), jnp.float32),
        ],
    )
    def k3(d2_hbm, ptsT_hbm, cxyz_hbm, t0_hbm, pg_out, mom_out,
           xb, yb, zb, cxb, cyb, czb, t0b, drowA, drowB, semA, semB,
           cand_v, cand_i, sel_idx, pgblock, mombuf):
        b = lax.axis_index("s") * NC + lax.axis_index("c")

        pltpu.sync_copy(ptsT_hbm.at[0, b], xb)
        pltpu.sync_copy(ptsT_hbm.at[1, b], yb)
        pltpu.sync_copy(ptsT_hbm.at[2, b], zb)
        pltpu.sync_copy(cxyz_hbm.at[0, b], cxb)
        pltpu.sync_copy(cxyz_hbm.at[1, b], cyb)
        pltpu.sync_copy(cxyz_hbm.at[2, b], czb)
        pltpu.sync_copy(t0_hbm.at[b], t0b)

        iota = lax.iota(jnp.int32, 16)
        zero16f = jnp.zeros((16,), jnp.float32)

        def process_row(r, drow, moms):
            rsp = jnp.full((16,), r, jnp.int32)
            t0s = plsc.load_gather(t0b, [rsp])
            cxs = plsc.load_gather(cxb, [rsp])
            cys = plsc.load_gather(cyb, [rsp])
            czs = plsc.load_gather(czb, [rsp])

            def scan_body(j, off):
                v = drow[pl.ds(j * 16, 16)]
                m = v <= t0s
                mi = m.astype(jnp.int32)
                pos = off + plsc.cumsum(mi) - 1
                pos = jnp.minimum(pos, CAND_CAP - 1)
                idxv = iota + j * 16
                plsc.store_scatter(cand_v, [pos], v, mask=m)
                plsc.store_scatter(cand_i, [pos], idxv, mask=m)
                return off + plsc.all_reduce_population_count(m)

            cnt_sp = lax.fori_loop(0, N // 16, scan_body,
                                   jnp.zeros((16,), jnp.int32), unroll=8)
            cnt = jnp.max(cnt_sp)
            nv4 = (cnt + 63) // 64

            def count_le(mid_sp):
                def cbody(c, acc):
                    for u in range(4):
                        ch = c * 4 + u
                        vbits = plsc.bitcast(cand_v[pl.ds(ch * 16, 16)],
                                             jnp.int32)
                        valid = (iota + ch * 16) < cnt_sp
                        le = jnp.logical_and(vbits <= mid_sp, valid)
                        acc = acc + plsc.all_reduce_population_count(le)
                    return acc
                return lax.fori_loop(0, nv4, cbody, jnp.zeros((16,), jnp.int32))

            def bis_body(_, st):
                lo, hi, cnt_lo = st
                mid = jnp.right_shift(lo + hi, 1)
                c = count_le(mid)
                ge = c >= 32
                hi = jnp.where(ge, mid, hi)
                new_lo = jnp.where(ge, lo, mid)
                cnt_lo = jnp.where(ge, cnt_lo, c)
                return new_lo, hi, cnt_lo

            lo0 = jnp.full((16,), jnp.int32(-2147483647), jnp.int32)
            hi0 = jnp.full((16,), jnp.int32(0x7F800000), jnp.int32)
            lo, hi, cnt_lo = lax.fori_loop(
                0, 32, bis_body, (lo0, hi0, jnp.zeros((16,), jnp.int32)))
            t32 = hi
            need_eq = 32 - cnt_lo

            def emit_body(c, st):
                soff, eqoff = st
                for u in range(4):
                    ch = c * 4 + u
                    vbits = plsc.bitcast(cand_v[pl.ds(ch * 16, 16)], jnp.int32)
                    ci = cand_i[pl.ds(ch * 16, 16)]
                    valid = (iota + ch * 16) < cnt_sp
                    m_lt = jnp.logical_and(vbits < t32, valid)
                    m_eq = jnp.logical_and(vbits == t32, valid)
                    eqrank = eqoff + plsc.cumsum(m_eq.astype(jnp.int32))
                    take_eq = jnp.logical_and(m_eq, eqrank <= need_eq)
                    m_sel = jnp.logical_or(m_lt, take_eq)
                    pos = soff + plsc.cumsum(m_sel.astype(jnp.int32)) - 1
                    pos = jnp.minimum(pos, 63)
                    plsc.store_scatter(sel_idx, [pos], ci, mask=m_sel)
                    soff = soff + plsc.all_reduce_population_count(m_sel)
                    eqoff = eqoff + plsc.all_reduce_population_count(m_eq)
                return soff, eqoff

            lax.fori_loop(0, nv4, emit_body,
                          (jnp.zeros((16,), jnp.int32),
                           jnp.zeros((16,), jnp.int32)))

            (ax, ay, az, axx, ayy, azz, axy, axz, ayz) = moms
            for s in range(2):
                gi = sel_idx[pl.ds(s * 16, 16)]
                gx = plsc.load_gather(xb, [gi]) - cxs
                gy = plsc.load_gather(yb, [gi]) - cys
                gz = plsc.load_gather(zb, [gi]) - czs
                base = r * (K * 3) + s * 48
                p0 = base + iota * 3
                plsc.store_scatter(pgblock, [p0], gx)
                plsc.store_scatter(pgblock, [p0 + 1], gy)
                plsc.store_scatter(pgblock, [p0 + 2], gz)
                ax = ax + gx
                ay = ay + gy
                az = az + gz
                axx = axx + gx * gx
                ayy = ayy + gy * gy
                azz = azz + gz * gz
                axy = axy + gx * gy
                axz = axz + gx * gz
                ayz = ayz + gy * gz
            return (ax, ay, az, axx, ayy, azz, axy, axz, ayz)

        def dma_start(r, buf, sem):
            pltpu.async_copy(d2_hbm.at[b * G + r], buf, sem)

        def dma_wait(r, buf, sem):
            pltpu.make_async_copy(d2_hbm.at[b * G + r], buf, sem).wait()

        dma_start(0, drowA, semA)

        def row_pair(rr, moms):
            r0 = rr * 2
            dma_wait(r0, drowA, semA)
            dma_start(r0 + 1, drowB, semB)
            moms = process_row(r0, drowA, moms)
            dma_wait(r0 + 1, drowB, semB)

            @pl.when(rr < G // 2 - 1)
            def _():
                dma_start(r0 + 2, drowA, semA)
            moms = process_row(r0 + 1, drowB, moms)
            return moms

        moms = lax.fori_loop(0, G // 2, row_pair,
                             tuple(zero16f for _ in range(9)))
        for k, acc in enumerate(moms):
            mombuf[pl.ds(k * 16, 16)] = acc
        pltpu.sync_copy(pgblock, pg_out.at[pl.ds(b * (G * K * 3), G * K * 3)])
        pltpu.sync_copy(mombuf, mom_out.at[b])

    pg_flat, mom = k3(d2, ptsT, cxyz, t0)
    return pg_flat.reshape(B * G * K, 3), mom


_R = 1024
_NSTEPS = (B * NUM_GROUP * GROUP_SIZE) // _R
_NPTS = float(B * NUM_GROUP * GROUP_SIZE)


def _k4b_body(pg_ref, mom_ref, W1_ref, b1_ref, g1_ref, be1_ref,
              W2_ref, b2_ref, W3_ref, b3_ref, f3_ref, stats_ref, acc_ref):
    i = pl.program_id(0)

    @pl.when(i == 0)
    def _():
        acc_ref[...] = jnp.zeros_like(acc_ref)

    momr = mom_ref[...]
    s = [jnp.sum(momr[:, k * 16:(k + 1) * 16]) for k in range(9)]
    mx, my, mz = s[0] / _NPTS, s[1] / _NPTS, s[2] / _NPTS
    sxx, syy, szz = s[3] / _NPTS, s[4] / _NPTS, s[5] / _NPTS
    sxy, sxz, syz = s[6] / _NPTS, s[7] / _NPTS, s[8] / _NPTS

    w = W1_ref[...]
    wx, wy, wz = w[:, 0], w[:, 1], w[:, 2]
    lin = wx * mx + wy * my + wz * mz
    quad = (wx * wx * sxx + wy * wy * syy + wz * wz * szz
            + 2.0 * (wx * wy * sxy + wx * wz * sxz + wy * wz * syz))
    var1 = quad - lin * lin
    mu1 = lin + b1_ref[...]
    s1 = g1_ref[...] / jnp.sqrt(var1 + 1e-5)
    t1 = be1_ref[...] - mu1 * s1

    pg = pg_ref[...]
    f1 = lax.dot_general(pg, w, (((1,), (1,)), ((), ())),
                         preferred_element_type=jnp.float32) + b1_ref[...]
    f1 = jax.nn.relu(f1 * s1 + t1)
    f2 = lax.dot_general(f1, W2_ref[...], (((1,), (1,)), ((), ())),
                         preferred_element_type=jnp.float32) + b2_ref[...]
    g = f2.reshape(_R // GROUP_SIZE, GROUP_SIZE, 256)
    fg = jnp.max(g, axis=1, keepdims=True)
    h = jnp.concatenate([jnp.broadcast_to(fg, g.shape), g], axis=-1)
    h = h.reshape(_R, 512)
    f3 = lax.dot_general(h, W3_ref[...], (((1,), (1,)), ((), ())),
                         preferred_element_type=jnp.float32) + b3_ref[...]
    f3_ref[...] = f3
    acc_ref[0, :] += jnp.sum(f3, axis=0)
    acc_ref[1, :] += jnp.sum(f3 * f3, axis=0)
    stats_ref[...] = acc_ref[...]


def _k4c_body(f3_ref, stats_ref, g2_ref, be2_ref, W4_ref, b4_ref, tok_ref):
    mu = stats_ref[0, :] / _NPTS
    var = stats_ref[1, :] / _NPTS - mu * mu
    s2 = g2_ref[...] / jnp.sqrt(var + 1e-5)
    t2 = be2_ref[...] - mu * s2
    r = jax.nn.relu(f3_ref[...] * s2 + t2)
    f4 = lax.dot_general(r, W4_ref[...], (((1,), (1,)), ((), ())),
                         preferred_element_type=jnp.float32) + b4_ref[...]
    tok_ref[...] = jnp.max(f4.reshape(_R // GROUP_SIZE, GROUP_SIZE, ENC_DIM),
                           axis=1)


def _k4(pg_rows, mom, W1, b1, g1, be1, W2, b2, W3, b3, g2, be2, W4, b4):
    full = lambda s: pl.BlockSpec(s, lambda i: tuple(0 for _ in s))
    f3, _stats = pl.pallas_call(
        _k4b_body,
        grid=(_NSTEPS,),
        in_specs=[
            pl.BlockSpec((_R, 3), lambda i: (i, 0)),
            full((B, 144)), full((128, 3)), full((128,)), full((128,)),
            full((128,)), full((256, 128)), full((256,)),
            full((512, 512)), full((512,)),
        ],
        out_specs=[
            pl.BlockSpec((_R, 512), lambda i: (i, 0)),
            pl.BlockSpec((2, 512), lambda i: (0, 0)),
        ],
        out_shape=[
            jax.ShapeDtypeStruct((B * NUM_GROUP * GROUP_SIZE, 512), jnp.float32),
            jax.ShapeDtypeStruct((2, 512), jnp.float32),
        ],
        scratch_shapes=[pltpu.VMEM((2, 512), jnp.float32)],
    )(pg_rows, mom, W1, b1, g1, be1, W2, b2, W3, b3)

    tokens = pl.pallas_call(
        _k4c_body,
        grid=(_NSTEPS,),
        in_specs=[
            pl.BlockSpec((_R, 512), lambda i: (i, 0)),
            full((2, 512)), full((512,)), full((512,)),
            full((ENC_DIM, 512)), full((ENC_DIM,)),
        ],
        out_specs=pl.BlockSpec((_R // GROUP_SIZE, ENC_DIM), lambda i: (i, 0)),
        out_shape=jax.ShapeDtypeStruct((B * NUM_GROUP, ENC_DIM), jnp.float32),
    )(f3, _stats, g2, be2, W4, b4)
    return tokens


def _bn(x, g, b):
    mu = x.mean(axis=(0, 1), keepdims=True)
    var = x.var(axis=(0, 1), keepdims=True)
    return (x - mu) / jnp.sqrt(var + 1e-5) * g + b


def _fps(xyz, K):
    B_, N_, _ = xyz.shape
    dists = jnp.full((B_, N_), 1e10, dtype=xyz.dtype)
    farthest = jnp.zeros((B_,), dtype=jnp.int32)
    idx_list = []
    for _ in range(K):
        idx_list.append(farthest)
        centroid = jnp.take_along_axis(xyz, farthest[:, None, None], axis=1)
        d = jnp.sum((xyz - centroid) ** 2, axis=-1)
        dists = jnp.minimum(dists, d)
        farthest = jnp.argmax(dists, axis=1).astype(jnp.int32)
    idxs = jnp.stack(idx_list, axis=1)
    centers = jnp.take_along_axis(xyz, idxs[:, :, None], axis=1)
    return centers, idxs


def _morton(grid, perm):
    x = grid[..., perm[0]]
    y = grid[..., perm[1]]
    z = grid[..., perm[2]]
    code = jnp.zeros(x.shape, dtype=jnp.int32)
    for b in range(BITS):
        code = code | (((x >> b) & 1) << (3 * b + 2)) | (((y >> b) & 1) << (3 * b + 1)) | (((z >> b) & 1) << (3 * b))
    return code


def _serialize(pos, feats, perm):
    grid = jnp.floor(pos / GRID_SIZE).astype(jnp.int32)
    grid = grid - grid.min(axis=1, keepdims=True)
    grid = jnp.clip(grid, 0, (1 << BITS) - 1)
    code = _morton(grid, perm)
    order = jnp.argsort(code, axis=1)
    return [jnp.take_along_axis(f, order[:, :, None], axis=1) for f in feats]


def _final_pallas(x, pos_all):
    Bb, T, D = x.shape

    def body(x_ref, p_ref, o_ref):
        xs = x_ref[0] + p_ref[0]
        cls = xs[T - 1, :]
        mean = jnp.mean(xs, axis=0)
        o_ref[0, 0] = jnp.concatenate([cls, mean], axis=-1)

    out = pl.pallas_call(
        body,
        grid=(Bb,),
        in_specs=[
            pl.BlockSpec((1, T, D), lambda b: (b, 0, 0)),
            pl.BlockSpec((1, T, D), lambda b: (b, 0, 0)),
        ],
        out_specs=pl.BlockSpec((1, 1, 2 * D), lambda b: (b, 0, 0)),
        out_shape=jax.ShapeDtypeStruct((Bb, 1, 2 * D), x.dtype),
    )(x, pos_all)
    return out.reshape(Bb, 2 * D)


def kernel(pts, W1, b1, g1, be1, W2, b2, W3, b3, g2, be2, W4, b4, Wp1, bp1, Wp2, bp2, gamma1, beta1, gamma2, beta2, cls_token, cls_pos):
    ptsT = jnp.transpose(pts, (2, 0, 1))
    cxyz = _k1(ptsT)
    centers = jnp.transpose(cxyz, (1, 2, 0))
    d2, t0 = _k2(ptsT, cxyz)
    pg_rows, mom = _k3_sc(d2, ptsT, cxyz, t0)
    tokens = _k4(pg_rows, mom, W1, b1, g1, be1, W2, b2, W3, b3,
                 g2, be2, W4, b4).reshape(B, NUM_GROUP, ENC_DIM)
    pos = jax.nn.gelu(centers @ Wp1.T + bp1) @ Wp2.T + bp2
    tok_f, pos_f = _serialize(centers, [tokens, pos], (0, 1, 2))
    tok_b, pos_b = _serialize(centers, [tokens, pos], (2, 1, 0))
    tok_f = tok_f * gamma1 + beta1
    tok_b = tok_b * gamma2 + beta2
    cls_t = jnp.broadcast_to(cls_token, (B, 1, TRANS_DIM))
    cls_p = jnp.broadcast_to(cls_pos, (B, 1, TRANS_DIM))
    pos_all = jnp.concatenate([pos_f, pos_b, cls_t], axis=1)
    x = jnp.concatenate([tok_f, tok_b, cls_p], axis=1)
    return _final_pallas(x, pos_all)


# speedup vs baseline: 9.9013x; 1.7324x over previous
import functools

import jax
import jax.numpy as jnp
from jax import lax
from jax.experimental import pallas as pl
from jax.experimental.pallas import tpu as pltpu
from jax.experimental.pallas import tpu_sc as plsc

TRANS_DIM = 384
ENC_DIM = 384
NUM_GROUP = 128
GROUP_SIZE = 32
GRID_SIZE = 0.02
BITS = 10

B = 32
N = 8192
NROWS = B * NUM_GROUP
CAND_CAP = 4096


def _k1_body(pts_ref, c_ref):
    x = pts_ref[0]
    y = pts_ref[1]
    z = pts_ref[2]
    iota_n = lax.broadcasted_iota(jnp.int32, (B, N), 1)
    iota_g = lax.broadcasted_iota(jnp.int32, (B, NUM_GROUP), 1)

    def body(k, st):
        dists, far, cxs, cys, czs = st
        oh = iota_n == far
        cx = jnp.sum(jnp.where(oh, x, 0.0), axis=1, keepdims=True)
        cy = jnp.sum(jnp.where(oh, y, 0.0), axis=1, keepdims=True)
        cz = jnp.sum(jnp.where(oh, z, 0.0), axis=1, keepdims=True)
        sel = iota_g == k
        cxs = jnp.where(sel, cx, cxs)
        cys = jnp.where(sel, cy, cys)
        czs = jnp.where(sel, cz, czs)
        dx = x - cx
        dy = y - cy
        dz = z - cz
        d = (dx * dx + dy * dy) + dz * dz
        dists = jnp.minimum(dists, d)
        m = jnp.max(dists, axis=1, keepdims=True)
        far = jnp.min(jnp.where(dists == m, iota_n, N), axis=1, keepdims=True)
        return dists, far, cxs, cys, czs

    init = (
        jnp.full((B, N), 1e10, jnp.float32),
        jnp.zeros((B, 1), jnp.int32),
        jnp.zeros((B, NUM_GROUP), jnp.float32),
        jnp.zeros((B, NUM_GROUP), jnp.float32),
        jnp.zeros((B, NUM_GROUP), jnp.float32),
    )
    _, _, cxs, cys, czs = lax.fori_loop(0, NUM_GROUP, body, init)
    c_ref[0] = cxs
    c_ref[1] = cys
    c_ref[2] = czs


def _k1(ptsT):
    return pl.pallas_call(
        _k1_body,
        in_specs=[pl.BlockSpec((3, B, N), lambda: (0, 0, 0))],
        out_specs=pl.BlockSpec((3, B, NUM_GROUP), lambda: (0, 0, 0)),
        out_shape=jax.ShapeDtypeStruct((3, B, NUM_GROUP), jnp.float32),
    )(ptsT)


def _k2_body(pts_ref, c_ref, d2_ref, t0_ref):
    pts_mat = pts_ref[:, 0, 0, :]
    c_mat = c_ref[:, 0, 0, :]
    x2 = pts_mat * pts_mat
    pp = (x2[0:1] + x2[1:2]) + x2[2:3]
    c2 = c_mat * c_mat
    cc = (c2[0:1] + c2[1:2]) + c2[2:3]
    prod = jax.lax.dot_general(
        c_mat, pts_mat, (((0,), (0,)), ((), ())),
        preferred_element_type=jnp.float32)
    d2 = (jnp.transpose(cc) + pp) - 2.0 * prod
    d2_ref[...] = d2
    m = jnp.min(d2.reshape(NUM_GROUP, 32, N // 32), axis=2)
    t0_ref[0, 0] = jnp.max(m, axis=1)


def _k2(ptsT, cxyz):
    d2, t0 = pl.pallas_call(
        _k2_body,
        grid=(B,),
        in_specs=[
            pl.BlockSpec((3, 1, 1, N), lambda b: (0, b, 0, 0)),
            pl.BlockSpec((3, 1, 1, NUM_GROUP), lambda b: (0, b, 0, 0)),
        ],
        out_specs=[
            pl.BlockSpec((NUM_GROUP, N), lambda b: (b, 0)),
            pl.BlockSpec((1, 1, NUM_GROUP), lambda b: (b, 0, 0)),
        ],
        out_shape=[
            jax.ShapeDtypeStruct((NROWS, N), jnp.float32),
            jax.ShapeDtypeStruct((B, 1, NUM_GROUP), jnp.float32),
        ],
    )(ptsT.reshape(3, B, 1, N), cxyz.reshape(3, B, 1, NUM_GROUP))
    return d2, t0.reshape(B, NUM_GROUP)


def _k3_sc(d2, ptsT, cxyz, t0):
    info = plsc.get_sparse_core_info()
    NC = info.num_cores

    mesh = plsc.VectorSubcoreMesh(core_axis_name="c", subcore_axis_name="s")
    G = NUM_GROUP
    K = GROUP_SIZE

    @functools.partial(
        pl.kernel,
        mesh=mesh,
        compiler_params=pltpu.CompilerParams(needs_layout_passes=False),
        out_type=[
            jax.ShapeDtypeStruct((B * G * K * 3,), jnp.float32),
            jax.ShapeDtypeStruct((B, 144), jnp.float32),
        ],
        scratch_types=[
            pltpu.VMEM((N,), jnp.float32),
            pltpu.VMEM((N,), jnp.float32),
            pltpu.VMEM((N,), jnp.float32),
            pltpu.VMEM((G,), jnp.float32),
            pltpu.VMEM((G,), jnp.float32),
            pltpu.VMEM((G,), jnp.float32),
            pltpu.VMEM((G,), jnp.float32),
            pltpu.VMEM((N,), jnp.float32),
            pltpu.VMEM((N,), jnp.float32),
            pltpu.SemaphoreType.DMA,
            pltpu.SemaphoreType.DMA,
            pltpu.VMEM((CAND_CAP + 64,), jnp.float32),
            pltpu.VMEM((CAND_CAP + 64,), jnp.int32),
            pltpu.VMEM((64,), jnp.int32),
            pltpu.VMEM((G * K * 3,), jnp.float32),
            pltpu.VMEM((144,), jnp.float32),
        ],
    )
    def k3(d2_hbm, ptsT_hbm, cxyz_hbm, t0_hbm, pg_out, mom_out,
           xb, yb, zb, cxb, cyb, czb, t0b, drowA, drowB, semA, semB,
           cand_v, cand_i, sel_idx, pgblock, mombuf):
        b = lax.axis_index("s") * NC + lax.axis_index("c")

        pltpu.sync_copy(ptsT_hbm.at[0, b], xb)
        pltpu.sync_copy(ptsT_hbm.at[1, b], yb)
        pltpu.sync_copy(ptsT_hbm.at[2, b], zb)
        pltpu.sync_copy(cxyz_hbm.at[0, b], cxb)
        pltpu.sync_copy(cxyz_hbm.at[1, b], cyb)
        pltpu.sync_copy(cxyz_hbm.at[2, b], czb)
        pltpu.sync_copy(t0_hbm.at[b], t0b)

        iota = lax.iota(jnp.int32, 16)
        zero16f = jnp.zeros((16,), jnp.float32)

        def process_row(r, drow, moms):
            rsp = jnp.full((16,), r, jnp.int32)
            t0s = plsc.load_gather(t0b, [rsp])
            cxs = plsc.load_gather(cxb, [rsp])
            cys = plsc.load_gather(cyb, [rsp])
            czs = plsc.load_gather(czb, [rsp])

            def scan_body(j, off):
                v = drow[pl.ds(j * 16, 16)]
                m = v <= t0s
                mi = m.astype(jnp.int32)
                pos = off + plsc.cumsum(mi) - 1
                pos = jnp.minimum(pos, CAND_CAP - 1)
                idxv = iota + j * 16
                plsc.store_scatter(cand_v, [pos], v, mask=m)
                plsc.store_scatter(cand_i, [pos], idxv, mask=m)
                return off + plsc.all_reduce_population_count(m)

            cnt_sp = plsc.parallel_loop(
                0, N // 16, carry=jnp.zeros((16,), jnp.int32), unroll=8
            )(scan_body)
            cnt = jnp.max(cnt_sp)
            nv4 = (cnt + 63) // 64

            def count_le(mid_sp):
                def cbody(c, acc):
                    for u in range(4):
                        ch = c * 4 + u
                        vbits = plsc.bitcast(cand_v[pl.ds(ch * 16, 16)],
                                             jnp.int32)
                        valid = (iota + ch * 16) < cnt_sp
                        le = jnp.logical_and(vbits <= mid_sp, valid)
                        acc = acc + plsc.all_reduce_population_count(le)
                    return acc
                return plsc.parallel_loop(
                    0, nv4, carry=jnp.zeros((16,), jnp.int32)
                )(cbody)

            def bis_body(_, st):
                lo, hi, cnt_lo = st
                mid = jnp.right_shift(lo + hi, 1)
                c = count_le(mid)
                ge = c >= 32
                hi = jnp.where(ge, mid, hi)
                new_lo = jnp.where(ge, lo, mid)
                cnt_lo = jnp.where(ge, cnt_lo, c)
                return new_lo, hi, cnt_lo

            lo0 = jnp.full((16,), jnp.int32(-2147483647), jnp.int32)
            hi0 = jnp.full((16,), jnp.int32(0x7F800000), jnp.int32)
            lo, hi, cnt_lo = lax.fori_loop(
                0, 32, bis_body, (lo0, hi0, jnp.zeros((16,), jnp.int32)))
            t32 = hi
            need_eq = 32 - cnt_lo

            def emit_body(c, st):
                soff, eqoff = st
                for u in range(4):
                    ch = c * 4 + u
                    vbits = plsc.bitcast(cand_v[pl.ds(ch * 16, 16)], jnp.int32)
                    ci = cand_i[pl.ds(ch * 16, 16)]
                    valid = (iota + ch * 16) < cnt_sp
                    m_lt = jnp.logical_and(vbits < t32, valid)
                    m_eq = jnp.logical_and(vbits == t32, valid)
                    eqrank = eqoff + plsc.cumsum(m_eq.astype(jnp.int32))
                    take_eq = jnp.logical_and(m_eq, eqrank <= need_eq)
                    m_sel = jnp.logical_or(m_lt, take_eq)
                    pos = soff + plsc.cumsum(m_sel.astype(jnp.int32)) - 1
                    pos = jnp.minimum(pos, 63)
                    plsc.store_scatter(sel_idx, [pos], ci, mask=m_sel)
                    soff = soff + plsc.all_reduce_population_count(m_sel)
                    eqoff = eqoff + plsc.all_reduce_population_count(m_eq)
                return soff, eqoff

            plsc.parallel_loop(
                0, nv4,
                carry=(jnp.zeros((16,), jnp.int32),
                       jnp.zeros((16,), jnp.int32)),
            )(emit_body)

            (ax, ay, az, axx, ayy, azz, axy, axz, ayz) = moms
            for s in range(2):
                gi = sel_idx[pl.ds(s * 16, 16)]
                gx = plsc.load_gather(xb, [gi]) - cxs
                gy = plsc.load_gather(yb, [gi]) - cys
                gz = plsc.load_gather(zb, [gi]) - czs
                base = r * (K * 3) + s * 48
                p0 = base + iota * 3
                plsc.store_scatter(pgblock, [p0], gx)
                plsc.store_scatter(pgblock, [p0 + 1], gy)
                plsc.store_scatter(pgblock, [p0 + 2], gz)
                ax = ax + gx
                ay = ay + gy
                az = az + gz
                axx = axx + gx * gx
                ayy = ayy + gy * gy
                azz = azz + gz * gz
                axy = axy + gx * gy
                axz = axz + gx * gz
                ayz = ayz + gy * gz
            return (ax, ay, az, axx, ayy, azz, axy, axz, ayz)

        def dma_start(r, buf, sem):
            pltpu.async_copy(d2_hbm.at[b * G + r], buf, sem)

        def dma_wait(r, buf, sem):
            pltpu.make_async_copy(d2_hbm.at[b * G + r], buf, sem).wait()

        dma_start(0, drowA, semA)

        def row_pair(rr, moms):
            r0 = rr * 2
            dma_wait(r0, drowA, semA)
            dma_start(r0 + 1, drowB, semB)
            moms = process_row(r0, drowA, moms)
            dma_wait(r0 + 1, drowB, semB)

            @pl.when(rr < G // 2 - 1)
            def _():
                dma_start(r0 + 2, drowA, semA)
            moms = process_row(r0 + 1, drowB, moms)
            return moms

        moms = lax.fori_loop(0, G // 2, row_pair,
                             tuple(zero16f for _ in range(9)))
        for k, acc in enumerate(moms):
            mombuf[pl.ds(k * 16, 16)] = acc
        pltpu.sync_copy(pgblock, pg_out.at[pl.ds(b * (G * K * 3), G * K * 3)])
        pltpu.sync_copy(mombuf, mom_out.at[b])

    pg_flat, mom = k3(d2, ptsT, cxyz, t0)
    return pg_flat.reshape(B * G * K, 3), mom


_R = 1024
_NSTEPS = (B * NUM_GROUP * GROUP_SIZE) // _R
_NPTS = float(B * NUM_GROUP * GROUP_SIZE)


def _k4b_body(pg_ref, mom_ref, W1_ref, b1_ref, g1_ref, be1_ref,
              W2_ref, b2_ref, W3_ref, b3_ref, f3_ref, stats_ref, acc_ref):
    i = pl.program_id(0)

    @pl.when(i == 0)
    def _():
        acc_ref[...] = jnp.zeros_like(acc_ref)

    momr = mom_ref[...]
    s = [jnp.sum(momr[:, k * 16:(k + 1) * 16]) for k in range(9)]
    mx, my, mz = s[0] / _NPTS, s[1] / _NPTS, s[2] / _NPTS
    sxx, syy, szz = s[3] / _NPTS, s[4] / _NPTS, s[5] / _NPTS
    sxy, sxz, syz = s[6] / _NPTS, s[7] / _NPTS, s[8] / _NPTS

    w = W1_ref[...]
    wx, wy, wz = w[:, 0], w[:, 1], w[:, 2]
    lin = wx * mx + wy * my + wz * mz
    quad = (wx * wx * sxx + wy * wy * syy + wz * wz * szz
            + 2.0 * (wx * wy * sxy + wx * wz * sxz + wy * wz * syz))
    var1 = quad - lin * lin
    mu1 = lin + b1_ref[...]
    s1 = g1_ref[...] / jnp.sqrt(var1 + 1e-5)
    t1 = be1_ref[...] - mu1 * s1

    pg = pg_ref[...]
    f1 = lax.dot_general(pg, w, (((1,), (1,)), ((), ())),
                         preferred_element_type=jnp.float32) + b1_ref[...]
    f1 = jax.nn.relu(f1 * s1 + t1)
    f2 = lax.dot_general(f1, W2_ref[...], (((1,), (1,)), ((), ())),
                         preferred_element_type=jnp.float32) + b2_ref[...]
    g = f2.reshape(_R // GROUP_SIZE, GROUP_SIZE, 256)
    fg = jnp.max(g, axis=1, keepdims=True)
    h = jnp.concatenate([jnp.broadcast_to(fg, g.shape), g], axis=-1)
    h = h.reshape(_R, 512)
    f3 = lax.dot_general(h, W3_ref[...], (((1,), (1,)), ((), ())),
                         preferred_element_type=jnp.float32) + b3_ref[...]
    f3_ref[...] = f3
    acc_ref[0, :] += jnp.sum(f3, axis=0)
    acc_ref[1, :] += jnp.sum(f3 * f3, axis=0)
    stats_ref[...] = acc_ref[...]


def _k4c_body(f3_ref, stats_ref, g2_ref, be2_ref, W4_ref, b4_ref, tok_ref):
    mu = stats_ref[0, :] / _NPTS
    var = stats_ref[1, :] / _NPTS - mu * mu
    s2 = g2_ref[...] / jnp.sqrt(var + 1e-5)
    t2 = be2_ref[...] - mu * s2
    r = jax.nn.relu(f3_ref[...] * s2 + t2)
    f4 = lax.dot_general(r, W4_ref[...], (((1,), (1,)), ((), ())),
                         preferred_element_type=jnp.float32) + b4_ref[...]
    tok_ref[...] = jnp.max(f4.reshape(_R // GROUP_SIZE, GROUP_SIZE, ENC_DIM),
                           axis=1)


def _k4(pg_rows, mom, W1, b1, g1, be1, W2, b2, W3, b3, g2, be2, W4, b4):
    full = lambda s: pl.BlockSpec(s, lambda i: tuple(0 for _ in s))
    f3, _stats = pl.pallas_call(
        _k4b_body,
        grid=(_NSTEPS,),
        in_specs=[
            pl.BlockSpec((_R, 3), lambda i: (i, 0)),
            full((B, 144)), full((128, 3)), full((128,)), full((128,)),
            full((128,)), full((256, 128)), full((256,)),
            full((512, 512)), full((512,)),
        ],
        out_specs=[
            pl.BlockSpec((_R, 512), lambda i: (i, 0)),
            pl.BlockSpec((2, 512), lambda i: (0, 0)),
        ],
        out_shape=[
            jax.ShapeDtypeStruct((B * NUM_GROUP * GROUP_SIZE, 512), jnp.float32),
            jax.ShapeDtypeStruct((2, 512), jnp.float32),
        ],
        scratch_shapes=[pltpu.VMEM((2, 512), jnp.float32)],
    )(pg_rows, mom, W1, b1, g1, be1, W2, b2, W3, b3)

    tokens = pl.pallas_call(
        _k4c_body,
        grid=(_NSTEPS,),
        in_specs=[
            pl.BlockSpec((_R, 512), lambda i: (i, 0)),
            full((2, 512)), full((512,)), full((512,)),
            full((ENC_DIM, 512)), full((ENC_DIM,)),
        ],
        out_specs=pl.BlockSpec((_R // GROUP_SIZE, ENC_DIM), lambda i: (i, 0)),
        out_shape=jax.ShapeDtypeStruct((B * NUM_GROUP, ENC_DIM), jnp.float32),
    )(f3, _stats, g2, be2, W4, b4)
    return tokens


def _bn(x, g, b):
    mu = x.mean(axis=(0, 1), keepdims=True)
    var = x.var(axis=(0, 1), keepdims=True)
    return (x - mu) / jnp.sqrt(var + 1e-5) * g + b


def _fps(xyz, K):
    B_, N_, _ = xyz.shape
    dists = jnp.full((B_, N_), 1e10, dtype=xyz.dtype)
    farthest = jnp.zeros((B_,), dtype=jnp.int32)
    idx_list = []
    for _ in range(K):
        idx_list.append(farthest)
        centroid = jnp.take_along_axis(xyz, farthest[:, None, None], axis=1)
        d = jnp.sum((xyz - centroid) ** 2, axis=-1)
        dists = jnp.minimum(dists, d)
        farthest = jnp.argmax(dists, axis=1).astype(jnp.int32)
    idxs = jnp.stack(idx_list, axis=1)
    centers = jnp.take_along_axis(xyz, idxs[:, :, None], axis=1)
    return centers, idxs


def _morton(grid, perm):
    x = grid[..., perm[0]]
    y = grid[..., perm[1]]
    z = grid[..., perm[2]]
    code = jnp.zeros(x.shape, dtype=jnp.int32)
    for b in range(BITS):
        code = code | (((x >> b) & 1) << (3 * b + 2)) | (((y >> b) & 1) << (3 * b + 1)) | (((z >> b) & 1) << (3 * b))
    return code


def _serialize(pos, feats, perm):
    grid = jnp.floor(pos / GRID_SIZE).astype(jnp.int32)
    grid = grid - grid.min(axis=1, keepdims=True)
    grid = jnp.clip(grid, 0, (1 << BITS) - 1)
    code = _morton(grid, perm)
    order = jnp.argsort(code, axis=1)
    return [jnp.take_along_axis(f, order[:, :, None], axis=1) for f in feats]


def _final_pallas(x, pos_all):
    Bb, T, D = x.shape

    def body(x_ref, p_ref, o_ref):
        xs = x_ref[0] + p_ref[0]
        cls = xs[T - 1, :]
        mean = jnp.mean(xs, axis=0)
        o_ref[0, 0] = jnp.concatenate([cls, mean], axis=-1)

    out = pl.pallas_call(
        body,
        grid=(Bb,),
        in_specs=[
            pl.BlockSpec((1, T, D), lambda b: (b, 0, 0)),
            pl.BlockSpec((1, T, D), lambda b: (b, 0, 0)),
        ],
        out_specs=pl.BlockSpec((1, 1, 2 * D), lambda b: (b, 0, 0)),
        out_shape=jax.ShapeDtypeStruct((Bb, 1, 2 * D), x.dtype),
    )(x, pos_all)
    return out.reshape(Bb, 2 * D)


def kernel(pts, W1, b1, g1, be1, W2, b2, W3, b3, g2, be2, W4, b4, Wp1, bp1, Wp2, bp2, gamma1, beta1, gamma2, beta2, cls_token, cls_pos):
    ptsT = jnp.transpose(pts, (2, 0, 1))
    cxyz = _k1(ptsT)
    centers = jnp.transpose(cxyz, (1, 2, 0))
    d2, t0 = _k2(ptsT, cxyz)
    pg_rows, mom = _k3_sc(d2, ptsT, cxyz, t0)
    tokens = _k4(pg_rows, mom, W1, b1, g1, be1, W2, b2, W3, b3,
                 g2, be2, W4, b4).reshape(B, NUM_GROUP, ENC_DIM)
    pos = jax.nn.gelu(centers @ Wp1.T + bp1) @ Wp2.T + bp2
    tok_f, pos_f = _serialize(centers, [tokens, pos], (0, 1, 2))
    tok_b, pos_b = _serialize(centers, [tokens, pos], (2, 1, 0))
    tok_f = tok_f * gamma1 + beta1
    tok_b = tok_b * gamma2 + beta2
    cls_t = jnp.broadcast_to(cls_token, (B, 1, TRANS_DIM))
    cls_p = jnp.broadcast_to(cls_pos, (B, 1, TRANS_DIM))
    pos_all = jnp.concatenate([pos_f, pos_b, cls_t], axis=1)
    x = jnp.concatenate([tok_f, tok_b, cls_p], axis=1)
    return _final_pallas(x, pos_all)


# speedup vs baseline: 10.2758x; 1.0378x over previous
import functools

import jax
import jax.numpy as jnp
from jax import lax
from jax.experimental import pallas as pl
from jax.experimental.pallas import tpu as pltpu
from jax.experimental.pallas import tpu_sc as plsc

TRANS_DIM = 384
ENC_DIM = 384
NUM_GROUP = 128
GROUP_SIZE = 32
GRID_SIZE = 0.02
BITS = 10

B = 32
N = 8192
NROWS = B * NUM_GROUP
CAND_CAP = 4096


def _k1_body(pts_ref, c_ref):
    x = pts_ref[0]
    y = pts_ref[1]
    z = pts_ref[2]
    iota_n = lax.broadcasted_iota(jnp.int32, (B, N), 1)
    iota_g = lax.broadcasted_iota(jnp.int32, (B, NUM_GROUP), 1)

    def body(k, st):
        dists, far, cxs, cys, czs = st
        oh = iota_n == far
        cx = jnp.sum(jnp.where(oh, x, 0.0), axis=1, keepdims=True)
        cy = jnp.sum(jnp.where(oh, y, 0.0), axis=1, keepdims=True)
        cz = jnp.sum(jnp.where(oh, z, 0.0), axis=1, keepdims=True)
        sel = iota_g == k
        cxs = jnp.where(sel, cx, cxs)
        cys = jnp.where(sel, cy, cys)
        czs = jnp.where(sel, cz, czs)
        dx = x - cx
        dy = y - cy
        dz = z - cz
        d = (dx * dx + dy * dy) + dz * dz
        dists = jnp.minimum(dists, d)
        m = jnp.max(dists, axis=1, keepdims=True)
        far = jnp.min(jnp.where(dists == m, iota_n, N), axis=1, keepdims=True)
        return dists, far, cxs, cys, czs

    init = (
        jnp.full((B, N), 1e10, jnp.float32),
        jnp.zeros((B, 1), jnp.int32),
        jnp.zeros((B, NUM_GROUP), jnp.float32),
        jnp.zeros((B, NUM_GROUP), jnp.float32),
        jnp.zeros((B, NUM_GROUP), jnp.float32),
    )
    _, _, cxs, cys, czs = lax.fori_loop(0, NUM_GROUP, body, init)
    c_ref[0] = cxs
    c_ref[1] = cys
    c_ref[2] = czs


def _k1(ptsT):
    return pl.pallas_call(
        _k1_body,
        in_specs=[pl.BlockSpec((3, B, N), lambda: (0, 0, 0))],
        out_specs=pl.BlockSpec((3, B, NUM_GROUP), lambda: (0, 0, 0)),
        out_shape=jax.ShapeDtypeStruct((3, B, NUM_GROUP), jnp.float32),
    )(ptsT)


def _k2_body(pts_ref, c_ref, d2_ref, t0_ref):
    pts_mat = pts_ref[:, 0, 0, :]
    c_mat = c_ref[:, 0, 0, :]
    x2 = pts_mat * pts_mat
    pp = (x2[0:1] + x2[1:2]) + x2[2:3]
    c2 = c_mat * c_mat
    cc = (c2[0:1] + c2[1:2]) + c2[2:3]
    prod = jax.lax.dot_general(
        c_mat, pts_mat, (((0,), (0,)), ((), ())),
        preferred_element_type=jnp.float32)
    d2 = (jnp.transpose(cc) + pp) - 2.0 * prod
    d2_ref[...] = d2
    m = jnp.min(d2.reshape(NUM_GROUP, 32, N // 32), axis=2)
    t0_ref[0, 0] = jnp.max(m, axis=1)


def _k2(ptsT, cxyz):
    d2, t0 = pl.pallas_call(
        _k2_body,
        grid=(B,),
        in_specs=[
            pl.BlockSpec((3, 1, 1, N), lambda b: (0, b, 0, 0)),
            pl.BlockSpec((3, 1, 1, NUM_GROUP), lambda b: (0, b, 0, 0)),
        ],
        out_specs=[
            pl.BlockSpec((NUM_GROUP, N), lambda b: (b, 0)),
            pl.BlockSpec((1, 1, NUM_GROUP), lambda b: (b, 0, 0)),
        ],
        out_shape=[
            jax.ShapeDtypeStruct((NROWS, N), jnp.float32),
            jax.ShapeDtypeStruct((B, 1, NUM_GROUP), jnp.float32),
        ],
    )(ptsT.reshape(3, B, 1, N), cxyz.reshape(3, B, 1, NUM_GROUP))
    return d2, t0.reshape(B, NUM_GROUP)


def _k3_sc(d2, ptsT, cxyz, t0):
    info = plsc.get_sparse_core_info()
    NC = info.num_cores

    mesh = plsc.VectorSubcoreMesh(core_axis_name="c", subcore_axis_name="s")
    G = NUM_GROUP
    K = GROUP_SIZE

    @functools.partial(
        pl.kernel,
        mesh=mesh,
        compiler_params=pltpu.CompilerParams(needs_layout_passes=False),
        out_type=[
            jax.ShapeDtypeStruct((B * G * K * 3,), jnp.float32),
            jax.ShapeDtypeStruct((B, 144), jnp.float32),
        ],
        scratch_types=[
            pltpu.VMEM((N,), jnp.float32),
            pltpu.VMEM((N,), jnp.float32),
            pltpu.VMEM((N,), jnp.float32),
            pltpu.VMEM((G,), jnp.float32),
            pltpu.VMEM((G,), jnp.float32),
            pltpu.VMEM((G,), jnp.float32),
            pltpu.VMEM((G,), jnp.float32),
            pltpu.VMEM((N,), jnp.float32),
            pltpu.VMEM((N,), jnp.float32),
            pltpu.SemaphoreType.DMA,
            pltpu.SemaphoreType.DMA,
            pltpu.VMEM((CAND_CAP + 64,), jnp.float32),
            pltpu.VMEM((CAND_CAP + 64,), jnp.int32),
            pltpu.VMEM((64,), jnp.int32),
            pltpu.VMEM((G * K * 3,), jnp.float32),
            pltpu.VMEM((144,), jnp.float32),
        ],
    )
    def k3(d2_hbm, ptsT_hbm, cxyz_hbm, t0_hbm, pg_out, mom_out,
           xb, yb, zb, cxb, cyb, czb, t0b, drowA, drowB, semA, semB,
           cand_v, cand_i, sel_idx, pgblock, mombuf):
        b = lax.axis_index("s") * NC + lax.axis_index("c")

        pltpu.sync_copy(ptsT_hbm.at[0, b], xb)
        pltpu.sync_copy(ptsT_hbm.at[1, b], yb)
        pltpu.sync_copy(ptsT_hbm.at[2, b], zb)
        pltpu.sync_copy(cxyz_hbm.at[0, b], cxb)
        pltpu.sync_copy(cxyz_hbm.at[1, b], cyb)
        pltpu.sync_copy(cxyz_hbm.at[2, b], czb)
        pltpu.sync_copy(t0_hbm.at[b], t0b)

        iota = lax.iota(jnp.int32, 16)
        zero16f = jnp.zeros((16,), jnp.float32)

        def process_row(r, drow, moms):
            rsp = jnp.full((16,), r, jnp.int32)
            t0s = plsc.load_gather(t0b, [rsp])
            cxs = plsc.load_gather(cxb, [rsp])
            cys = plsc.load_gather(cyb, [rsp])
            czs = plsc.load_gather(czb, [rsp])

            def scan_body(j, off):
                v = drow[pl.ds(j * 16, 16)]
                m = v <= t0s
                mi = m.astype(jnp.int32)
                pos = off + plsc.cumsum(mi) - 1
                pos = jnp.minimum(pos, CAND_CAP - 1)
                idxv = iota + j * 16
                plsc.store_scatter(cand_v, [pos], v, mask=m)
                plsc.store_scatter(cand_i, [pos], idxv, mask=m)
                return off + plsc.all_reduce_population_count(m)

            cnt_sp = plsc.parallel_loop(
                0, N // 16, carry=jnp.zeros((16,), jnp.int32), unroll=8
            )(scan_body)
            cnt = jnp.max(cnt_sp)
            nv4 = (cnt + 63) // 64

            def count_le(mid_sp):
                def cbody(c, acc):
                    for u in range(4):
                        ch = c * 4 + u
                        vbits = plsc.bitcast(cand_v[pl.ds(ch * 16, 16)],
                                             jnp.int32)
                        valid = (iota + ch * 16) < cnt_sp
                        le = jnp.logical_and(vbits <= mid_sp, valid)
                        acc = acc + plsc.all_reduce_population_count(le)
                    return acc
                return plsc.parallel_loop(
                    0, nv4, carry=jnp.zeros((16,), jnp.int32)
                )(cbody)

            def bis_body(_, st):
                lo, hi, cnt_lo = st
                mid = jnp.right_shift(lo + hi, 1)
                c = count_le(mid)
                ge = c >= 32
                hi = jnp.where(ge, mid, hi)
                new_lo = jnp.where(ge, lo, mid)
                cnt_lo = jnp.where(ge, cnt_lo, c)
                return new_lo, hi, cnt_lo

            lo0 = jnp.full((16,), jnp.int32(-2147483647), jnp.int32)
            hi0 = jnp.full((16,), jnp.int32(0x7F800000), jnp.int32)
            lo, hi, cnt_lo = lax.fori_loop(
                0, 32, bis_body, (lo0, hi0, jnp.zeros((16,), jnp.int32)))
            t32 = hi
            need_eq = 32 - cnt_lo

            def emit_body(c, st):
                soff, eqoff = st
                for u in range(4):
                    ch = c * 4 + u
                    vbits = plsc.bitcast(cand_v[pl.ds(ch * 16, 16)], jnp.int32)
                    ci = cand_i[pl.ds(ch * 16, 16)]
                    valid = (iota + ch * 16) < cnt_sp
                    m_lt = jnp.logical_and(vbits < t32, valid)
                    m_eq = jnp.logical_and(vbits == t32, valid)
                    eqrank = eqoff + plsc.cumsum(m_eq.astype(jnp.int32))
                    take_eq = jnp.logical_and(m_eq, eqrank <= need_eq)
                    m_sel = jnp.logical_or(m_lt, take_eq)
                    pos = soff + plsc.cumsum(m_sel.astype(jnp.int32)) - 1
                    pos = jnp.minimum(pos, 63)
                    plsc.store_scatter(sel_idx, [pos], ci, mask=m_sel)
                    soff = soff + plsc.all_reduce_population_count(m_sel)
                    eqoff = eqoff + plsc.all_reduce_population_count(m_eq)
                return soff, eqoff

            plsc.parallel_loop(
                0, nv4,
                carry=(jnp.zeros((16,), jnp.int32),
                       jnp.zeros((16,), jnp.int32)),
            )(emit_body)

            (ax, ay, az, axx, ayy, azz, axy, axz, ayz) = moms
            for s in range(2):
                gi = sel_idx[pl.ds(s * 16, 16)]
                gx = plsc.load_gather(xb, [gi]) - cxs
                gy = plsc.load_gather(yb, [gi]) - cys
                gz = plsc.load_gather(zb, [gi]) - czs
                base = r * (K * 3) + s * 48
                p0 = base + iota * 3
                plsc.store_scatter(pgblock, [p0], gx)
                plsc.store_scatter(pgblock, [p0 + 1], gy)
                plsc.store_scatter(pgblock, [p0 + 2], gz)
                ax = ax + gx
                ay = ay + gy
                az = az + gz
                axx = axx + gx * gx
                ayy = ayy + gy * gy
                azz = azz + gz * gz
                axy = axy + gx * gy
                axz = axz + gx * gz
                ayz = ayz + gy * gz
            return (ax, ay, az, axx, ayy, azz, axy, axz, ayz)

        def dma_start(r, buf, sem):
            pltpu.async_copy(d2_hbm.at[b * G + r], buf, sem)

        def dma_wait(r, buf, sem):
            pltpu.make_async_copy(d2_hbm.at[b * G + r], buf, sem).wait()

        dma_start(0, drowA, semA)

        def row_pair(rr, moms):
            r0 = rr * 2
            dma_wait(r0, drowA, semA)
            dma_start(r0 + 1, drowB, semB)
            moms = process_row(r0, drowA, moms)
            dma_wait(r0 + 1, drowB, semB)

            @pl.when(rr < G // 2 - 1)
            def _():
                dma_start(r0 + 2, drowA, semA)
            moms = process_row(r0 + 1, drowB, moms)
            return moms

        moms = lax.fori_loop(0, G // 2, row_pair,
                             tuple(zero16f for _ in range(9)))
        for k, acc in enumerate(moms):
            mombuf[pl.ds(k * 16, 16)] = acc
        pltpu.sync_copy(pgblock, pg_out.at[pl.ds(b * (G * K * 3), G * K * 3)])
        pltpu.sync_copy(mombuf, mom_out.at[b])

    pg_flat, mom = k3(d2, ptsT, cxyz, t0)
    return pg_flat.reshape(B * G * K, 3), mom


_R = 1024
_NSTEPS = (B * NUM_GROUP * GROUP_SIZE) // _R
_NPTS = float(B * NUM_GROUP * GROUP_SIZE)


def _k4b_body(pg_ref, mom_ref, W1_ref, b1_ref, g1_ref, be1_ref,
              W2_ref, b2_ref, W3_ref, b3_ref, f3_ref, stats_ref, acc_ref):
    i = pl.program_id(0)

    @pl.when(i == 0)
    def _():
        acc_ref[...] = jnp.zeros_like(acc_ref)

    momr = mom_ref[...]
    s = [jnp.sum(momr[:, k * 16:(k + 1) * 16]) for k in range(9)]
    mx, my, mz = s[0] / _NPTS, s[1] / _NPTS, s[2] / _NPTS
    sxx, syy, szz = s[3] / _NPTS, s[4] / _NPTS, s[5] / _NPTS
    sxy, sxz, syz = s[6] / _NPTS, s[7] / _NPTS, s[8] / _NPTS

    w = W1_ref[...]
    wx, wy, wz = w[:, 0], w[:, 1], w[:, 2]
    lin = wx * mx + wy * my + wz * mz
    quad = (wx * wx * sxx + wy * wy * syy + wz * wz * szz
            + 2.0 * (wx * wy * sxy + wx * wz * sxz + wy * wz * syz))
    var1 = quad - lin * lin
    mu1 = lin + b1_ref[...]
    s1 = g1_ref[...] / jnp.sqrt(var1 + 1e-5)
    t1 = be1_ref[...] - mu1 * s1

    pg = pg_ref[...]
    f1 = lax.dot_general(pg, w, (((1,), (1,)), ((), ())),
                         preferred_element_type=jnp.float32) + b1_ref[...]
    f1 = jax.nn.relu(f1 * s1 + t1)
    f2 = lax.dot_general(f1, W2_ref[...], (((1,), (1,)), ((), ())),
                         preferred_element_type=jnp.float32) + b2_ref[...]
    g = f2.reshape(_R // GROUP_SIZE, GROUP_SIZE, 256)
    fg = jnp.max(g, axis=1, keepdims=True)
    h = jnp.concatenate([jnp.broadcast_to(fg, g.shape), g], axis=-1)
    h = h.reshape(_R, 512)
    f3 = lax.dot_general(h, W3_ref[...], (((1,), (1,)), ((), ())),
                         preferred_element_type=jnp.float32) + b3_ref[...]
    f3_ref[...] = f3
    acc_ref[0, :] += jnp.sum(f3, axis=0)
    acc_ref[1, :] += jnp.sum(f3 * f3, axis=0)
    stats_ref[...] = acc_ref[...]


def _k4c_body(f3_ref, stats_ref, g2_ref, be2_ref, W4_ref, b4_ref, tok_ref):
    mu = stats_ref[0, :] / _NPTS
    var = stats_ref[1, :] / _NPTS - mu * mu
    s2 = g2_ref[...] / jnp.sqrt(var + 1e-5)
    t2 = be2_ref[...] - mu * s2
    r = jax.nn.relu(f3_ref[...] * s2 + t2)
    f4 = lax.dot_general(r, W4_ref[...], (((1,), (1,)), ((), ())),
                         preferred_element_type=jnp.float32) + b4_ref[...]
    tok_ref[...] = jnp.max(f4.reshape(_R // GROUP_SIZE, GROUP_SIZE, ENC_DIM),
                           axis=1)


def _k4(pg_rows, mom, W1, b1, g1, be1, W2, b2, W3, b3, g2, be2, W4, b4):
    full = lambda s: pl.BlockSpec(s, lambda i: tuple(0 for _ in s))
    f3, _stats = pl.pallas_call(
        _k4b_body,
        grid=(_NSTEPS,),
        in_specs=[
            pl.BlockSpec((_R, 3), lambda i: (i, 0)),
            full((B, 144)), full((128, 3)), full((128,)), full((128,)),
            full((128,)), full((256, 128)), full((256,)),
            full((512, 512)), full((512,)),
        ],
        out_specs=[
            pl.BlockSpec((_R, 512), lambda i: (i, 0)),
            pl.BlockSpec((2, 512), lambda i: (0, 0)),
        ],
        out_shape=[
            jax.ShapeDtypeStruct((B * NUM_GROUP * GROUP_SIZE, 512), jnp.float32),
            jax.ShapeDtypeStruct((2, 512), jnp.float32),
        ],
        scratch_shapes=[pltpu.VMEM((2, 512), jnp.float32)],
    )(pg_rows, mom, W1, b1, g1, be1, W2, b2, W3, b3)

    tokens = pl.pallas_call(
        _k4c_body,
        grid=(_NSTEPS,),
        in_specs=[
            pl.BlockSpec((_R, 512), lambda i: (i, 0)),
            full((2, 512)), full((512,)), full((512,)),
            full((ENC_DIM, 512)), full((ENC_DIM,)),
        ],
        out_specs=pl.BlockSpec((_R // GROUP_SIZE, ENC_DIM), lambda i: (i, 0)),
        out_shape=jax.ShapeDtypeStruct((B * NUM_GROUP, ENC_DIM), jnp.float32),
    )(f3, _stats, g2, be2, W4, b4)
    return tokens


def _k5_body(tok_ref, c_ref, Wp1_ref, bp1_ref, Wp2_ref, bp2_ref,
             ga1_ref, be1_ref, ga2_ref, be2_ref, clst_ref, clsp_ref, o_ref):
    G = NUM_GROUP
    c = c_ref[:, 0, 0, :]
    tok = tok_ref[0]
    ct = jnp.transpose(c)
    h1 = lax.dot_general(ct, Wp1_ref[...], (((1,), (1,)), ((), ())),
                         preferred_element_type=jnp.float32) + bp1_ref[...]
    pos = lax.dot_general(jax.nn.gelu(h1), Wp2_ref[...],
                          (((1,), (1,)), ((), ())),
                          preferred_element_type=jnp.float32) + bp2_ref[...]

    grid = jnp.floor(c * (1.0 / GRID_SIZE)).astype(jnp.int32)
    grid = grid - jnp.min(grid, axis=1, keepdims=True)
    grid = jnp.clip(grid, 0, (1 << BITS) - 1)

    def morton(x, y, z):
        code = jnp.zeros_like(x)
        for bb in range(BITS):
            code = (code
                    | (((x >> bb) & 1) << (3 * bb + 2))
                    | (((y >> bb) & 1) << (3 * bb + 1))
                    | (((z >> bb) & 1) << (3 * bb)))
        return code

    gx, gy, gz = grid[0:1], grid[1:2], grid[2:3]
    code_f = morton(gx, gy, gz)
    code_b = morton(gz, gy, gx)

    ii = lax.broadcasted_iota(jnp.int32, (G, G), 1)
    jj = lax.broadcasted_iota(jnp.int32, (G, G), 0)
    iota_d = jj

    def perm_matrix(code):
        cr = jnp.transpose(code)
        lt = code < cr
        eqm = jnp.logical_and(code == cr, ii < jj)
        rank = jnp.sum(jnp.logical_or(lt, eqm).astype(jnp.int32),
                       axis=1, keepdims=True)
        return (iota_d == jnp.transpose(rank)).astype(jnp.float32)

    Pf = perm_matrix(code_f)
    Pb = perm_matrix(code_b)

    def apply(P, m):
        return lax.dot_general(P, m, (((1,), (0,)), ((), ())),
                               preferred_element_type=jnp.float32)

    tok_f = apply(Pf, tok) * ga1_ref[...] + be1_ref[...]
    pos_f = apply(Pf, pos)
    tok_b = apply(Pb, tok) * ga2_ref[...] + be2_ref[...]
    pos_b = apply(Pb, pos)

    cls_row = clsp_ref[0, 0, :] + clst_ref[0, 0, :]
    xs = (tok_f + pos_f)
    xb = (tok_b + pos_b)
    total = jnp.sum(xs, axis=0) + jnp.sum(xb, axis=0) + cls_row
    mean = total * (1.0 / (2 * G + 1))
    o_ref[0, 0] = jnp.concatenate([cls_row, mean], axis=-1)


def _k5(tokens, cxyz, Wp1, bp1, Wp2, bp2, gamma1, beta1, gamma2, beta2,
        cls_token, cls_pos):
    G = NUM_GROUP
    full = lambda s: pl.BlockSpec(s, lambda b: tuple(0 for _ in s))
    out = pl.pallas_call(
        _k5_body,
        grid=(B,),
        in_specs=[
            pl.BlockSpec((1, G, TRANS_DIM), lambda b: (b, 0, 0)),
            pl.BlockSpec((3, 1, 1, G), lambda b: (0, b, 0, 0)),
            full((128, 3)), full((128,)), full((TRANS_DIM, 128)),
            full((TRANS_DIM,)),
            full((TRANS_DIM,)), full((TRANS_DIM,)),
            full((TRANS_DIM,)), full((TRANS_DIM,)),
            full((1, 1, TRANS_DIM)), full((1, 1, TRANS_DIM)),
        ],
        out_specs=pl.BlockSpec((1, 1, 2 * TRANS_DIM), lambda b: (b, 0, 0)),
        out_shape=jax.ShapeDtypeStruct((B, 1, 2 * TRANS_DIM), jnp.float32),
    )(tokens, cxyz.reshape(3, B, 1, G), Wp1, bp1, Wp2, bp2,
      gamma1, beta1, gamma2, beta2, cls_token, cls_pos)
    return out.reshape(B, 2 * TRANS_DIM)


def kernel(pts, W1, b1, g1, be1, W2, b2, W3, b3, g2, be2, W4, b4, Wp1, bp1, Wp2, bp2, gamma1, beta1, gamma2, beta2, cls_token, cls_pos):
    ptsT = jnp.transpose(pts, (2, 0, 1))
    cxyz = _k1(ptsT)
    d2, t0 = _k2(ptsT, cxyz)
    pg_rows, mom = _k3_sc(d2, ptsT, cxyz, t0)
    tokens = _k4(pg_rows, mom, W1, b1, g1, be1, W2, b2, W3, b3,
                 g2, be2, W4, b4).reshape(B, NUM_GROUP, ENC_DIM)
    return _k5(tokens, cxyz, Wp1, bp1, Wp2, bp2,
               gamma1, beta1, gamma2, beta2, cls_token, cls_pos)


# speedup vs baseline: 10.5111x; 1.0229x over previous
import functools

import jax
import jax.numpy as jnp
from jax import lax
from jax.experimental import pallas as pl
from jax.experimental.pallas import tpu as pltpu
from jax.experimental.pallas import tpu_sc as plsc

TRANS_DIM = 384
ENC_DIM = 384
NUM_GROUP = 128
GROUP_SIZE = 32
GRID_SIZE = 0.02
BITS = 10

B = 32
N = 8192
NROWS = B * NUM_GROUP
CAND_CAP = 4096


def _k1_body(pts_ref, c_ref):
    x = pts_ref[0]
    y = pts_ref[1]
    z = pts_ref[2]
    iota_n = lax.broadcasted_iota(jnp.int32, (B, N), 1)
    iota_g = lax.broadcasted_iota(jnp.int32, (B, NUM_GROUP), 1)

    def body(k, st):
        dists, far, cxs, cys, czs = st
        oh = iota_n == far
        cx = jnp.sum(jnp.where(oh, x, 0.0), axis=1, keepdims=True)
        cy = jnp.sum(jnp.where(oh, y, 0.0), axis=1, keepdims=True)
        cz = jnp.sum(jnp.where(oh, z, 0.0), axis=1, keepdims=True)
        sel = iota_g == k
        cxs = jnp.where(sel, cx, cxs)
        cys = jnp.where(sel, cy, cys)
        czs = jnp.where(sel, cz, czs)
        dx = x - cx
        dy = y - cy
        dz = z - cz
        d = (dx * dx + dy * dy) + dz * dz
        dists = jnp.minimum(dists, d)
        m = jnp.max(dists, axis=1, keepdims=True)
        far = jnp.min(jnp.where(dists == m, iota_n, N), axis=1, keepdims=True)
        return dists, far, cxs, cys, czs

    init = (
        jnp.full((B, N), 1e10, jnp.float32),
        jnp.zeros((B, 1), jnp.int32),
        jnp.zeros((B, NUM_GROUP), jnp.float32),
        jnp.zeros((B, NUM_GROUP), jnp.float32),
        jnp.zeros((B, NUM_GROUP), jnp.float32),
    )
    _, _, cxs, cys, czs = lax.fori_loop(0, NUM_GROUP, body, init)
    c_ref[0] = cxs
    c_ref[1] = cys
    c_ref[2] = czs


def _k1(ptsT):
    return pl.pallas_call(
        _k1_body,
        in_specs=[pl.BlockSpec((3, B, N), lambda: (0, 0, 0))],
        out_specs=pl.BlockSpec((3, B, NUM_GROUP), lambda: (0, 0, 0)),
        out_shape=jax.ShapeDtypeStruct((3, B, NUM_GROUP), jnp.float32),
    )(ptsT)


def _k2_body(pts_ref, c_ref, d2_ref, t0_ref):
    pts_mat = pts_ref[:, 0, 0, :]
    c_mat = c_ref[:, 0, 0, :]
    x2 = pts_mat * pts_mat
    pp = (x2[0:1] + x2[1:2]) + x2[2:3]
    c2 = c_mat * c_mat
    cc = (c2[0:1] + c2[1:2]) + c2[2:3]
    prod = jax.lax.dot_general(
        c_mat, pts_mat, (((0,), (0,)), ((), ())),
        preferred_element_type=jnp.float32)
    d2 = (jnp.transpose(cc) + pp) - 2.0 * prod
    d2_ref[...] = d2
    m = jnp.min(d2.reshape(NUM_GROUP, 32, N // 32), axis=2)
    t0_ref[0, 0] = jnp.max(m, axis=1)


def _k2(ptsT, cxyz):
    d2, t0 = pl.pallas_call(
        _k2_body,
        grid=(B,),
        in_specs=[
            pl.BlockSpec((3, 1, 1, N), lambda b: (0, b, 0, 0)),
            pl.BlockSpec((3, 1, 1, NUM_GROUP), lambda b: (0, b, 0, 0)),
        ],
        out_specs=[
            pl.BlockSpec((NUM_GROUP, N), lambda b: (b, 0)),
            pl.BlockSpec((1, 1, NUM_GROUP), lambda b: (b, 0, 0)),
        ],
        out_shape=[
            jax.ShapeDtypeStruct((NROWS, N), jnp.float32),
            jax.ShapeDtypeStruct((B, 1, NUM_GROUP), jnp.float32),
        ],
    )(ptsT.reshape(3, B, 1, N), cxyz.reshape(3, B, 1, NUM_GROUP))
    return d2, t0.reshape(B, NUM_GROUP)


def _k3_sc(d2, ptsT, cxyz, t0):
    info = plsc.get_sparse_core_info()
    NC = info.num_cores

    mesh = plsc.VectorSubcoreMesh(core_axis_name="c", subcore_axis_name="s")
    G = NUM_GROUP
    K = GROUP_SIZE

    @functools.partial(
        pl.kernel,
        mesh=mesh,
        compiler_params=pltpu.CompilerParams(needs_layout_passes=False),
        out_type=[
            jax.ShapeDtypeStruct((B * G * K * 3,), jnp.float32),
            jax.ShapeDtypeStruct((B, 144), jnp.float32),
        ],
        scratch_types=[
            pltpu.VMEM((N,), jnp.float32),
            pltpu.VMEM((N,), jnp.float32),
            pltpu.VMEM((N,), jnp.float32),
            pltpu.VMEM((G,), jnp.float32),
            pltpu.VMEM((G,), jnp.float32),
            pltpu.VMEM((G,), jnp.float32),
            pltpu.VMEM((G,), jnp.float32),
            pltpu.VMEM((N,), jnp.float32),
            pltpu.VMEM((N,), jnp.float32),
            pltpu.SemaphoreType.DMA,
            pltpu.SemaphoreType.DMA,
            pltpu.VMEM((CAND_CAP + 64,), jnp.float32),
            pltpu.VMEM((CAND_CAP + 64,), jnp.int32),
            pltpu.VMEM((64,), jnp.int32),
            pltpu.VMEM((G * K * 3,), jnp.float32),
            pltpu.VMEM((144,), jnp.float32),
        ],
    )
    def k3(d2_hbm, ptsT_hbm, cxyz_hbm, t0_hbm, pg_out, mom_out,
           xb, yb, zb, cxb, cyb, czb, t0b, drowA, drowB, semA, semB,
           cand_v, cand_i, sel_idx, pgblock, mombuf):
        b = lax.axis_index("s") * NC + lax.axis_index("c")

        pltpu.sync_copy(ptsT_hbm.at[0, b], xb)
        pltpu.sync_copy(ptsT_hbm.at[1, b], yb)
        pltpu.sync_copy(ptsT_hbm.at[2, b], zb)
        pltpu.sync_copy(cxyz_hbm.at[0, b], cxb)
        pltpu.sync_copy(cxyz_hbm.at[1, b], cyb)
        pltpu.sync_copy(cxyz_hbm.at[2, b], czb)
        pltpu.sync_copy(t0_hbm.at[b], t0b)

        iota = lax.iota(jnp.int32, 16)
        zero16f = jnp.zeros((16,), jnp.float32)

        def process_row(r, drow, moms):
            rsp = jnp.full((16,), r, jnp.int32)
            t0s = plsc.load_gather(t0b, [rsp])
            cxs = plsc.load_gather(cxb, [rsp])
            cys = plsc.load_gather(cyb, [rsp])
            czs = plsc.load_gather(czb, [rsp])

            def scan_body(j, off):
                v = drow[pl.ds(j * 16, 16)]
                m = v <= t0s
                mi = m.astype(jnp.int32)
                pos = off + plsc.cumsum(mi) - 1
                pos = jnp.minimum(pos, CAND_CAP - 1)
                idxv = iota + j * 16
                plsc.store_scatter(cand_v, [pos], v, mask=m)
                plsc.store_scatter(cand_i, [pos], idxv, mask=m)
                return off + plsc.all_reduce_population_count(m)

            cnt_sp = plsc.parallel_loop(
                0, N // 16, carry=jnp.zeros((16,), jnp.int32), unroll=8
            )(scan_body)
            cnt = jnp.max(cnt_sp)
            nv4 = (cnt + 63) // 64

            def count_le(mid_sp):
                def cbody(c, acc):
                    for u in range(4):
                        ch = c * 4 + u
                        vbits = plsc.bitcast(cand_v[pl.ds(ch * 16, 16)],
                                             jnp.int32)
                        valid = (iota + ch * 16) < cnt_sp
                        le = jnp.logical_and(vbits <= mid_sp, valid)
                        acc = acc + plsc.all_reduce_population_count(le)
                    return acc
                return plsc.parallel_loop(
                    0, nv4, carry=jnp.zeros((16,), jnp.int32)
                )(cbody)

            def bis_body(_, st):
                lo, hi, cnt_lo = st
                mid = jnp.right_shift(lo + hi, 1)
                c = count_le(mid)
                ge = c >= 32
                hi = jnp.where(ge, mid, hi)
                new_lo = jnp.where(ge, lo, mid)
                cnt_lo = jnp.where(ge, cnt_lo, c)
                return new_lo, hi, cnt_lo

            lo0 = jnp.full((16,), jnp.int32(-2147483647), jnp.int32)
            hi0 = jnp.full((16,), jnp.int32(0x7F800000), jnp.int32)
            lo, hi, cnt_lo = lax.fori_loop(
                0, 32, bis_body, (lo0, hi0, jnp.zeros((16,), jnp.int32)))
            t32 = hi
            need_eq = 32 - cnt_lo

            def emit_body(c, st):
                soff, eqoff = st
                for u in range(4):
                    ch = c * 4 + u
                    vbits = plsc.bitcast(cand_v[pl.ds(ch * 16, 16)], jnp.int32)
                    ci = cand_i[pl.ds(ch * 16, 16)]
                    valid = (iota + ch * 16) < cnt_sp
                    m_lt = jnp.logical_and(vbits < t32, valid)
                    m_eq = jnp.logical_and(vbits == t32, valid)
                    eqrank = eqoff + plsc.cumsum(m_eq.astype(jnp.int32))
                    take_eq = jnp.logical_and(m_eq, eqrank <= need_eq)
                    m_sel = jnp.logical_or(m_lt, take_eq)
                    pos = soff + plsc.cumsum(m_sel.astype(jnp.int32)) - 1
                    pos = jnp.minimum(pos, 63)
                    plsc.store_scatter(sel_idx, [pos], ci, mask=m_sel)
                    soff = soff + plsc.all_reduce_population_count(m_sel)
                    eqoff = eqoff + plsc.all_reduce_population_count(m_eq)
                return soff, eqoff

            plsc.parallel_loop(
                0, nv4,
                carry=(jnp.zeros((16,), jnp.int32),
                       jnp.zeros((16,), jnp.int32)),
            )(emit_body)

            (ax, ay, az, axx, ayy, azz, axy, axz, ayz) = moms
            for s in range(2):
                gi = sel_idx[pl.ds(s * 16, 16)]
                gx = plsc.load_gather(xb, [gi]) - cxs
                gy = plsc.load_gather(yb, [gi]) - cys
                gz = plsc.load_gather(zb, [gi]) - czs
                base = r * (K * 3) + s * 48
                p0 = base + iota * 3
                plsc.store_scatter(pgblock, [p0], gx)
                plsc.store_scatter(pgblock, [p0 + 1], gy)
                plsc.store_scatter(pgblock, [p0 + 2], gz)
                ax = ax + gx
                ay = ay + gy
                az = az + gz
                axx = axx + gx * gx
                ayy = ayy + gy * gy
                azz = azz + gz * gz
                axy = axy + gx * gy
                axz = axz + gx * gz
                ayz = ayz + gy * gz
            return (ax, ay, az, axx, ayy, azz, axy, axz, ayz)

        def dma_start(r, buf, sem):
            pltpu.async_copy(d2_hbm.at[b * G + r], buf, sem)

        def dma_wait(r, buf, sem):
            pltpu.make_async_copy(d2_hbm.at[b * G + r], buf, sem).wait()

        dma_start(0, drowA, semA)

        def row_pair(rr, moms):
            r0 = rr * 2
            dma_wait(r0, drowA, semA)
            dma_start(r0 + 1, drowB, semB)
            moms = process_row(r0, drowA, moms)
            dma_wait(r0 + 1, drowB, semB)

            @pl.when(rr < G // 2 - 1)
            def _():
                dma_start(r0 + 2, drowA, semA)
            moms = process_row(r0 + 1, drowB, moms)
            return moms

        moms = lax.fori_loop(0, G // 2, row_pair,
                             tuple(zero16f for _ in range(9)))
        for k, acc in enumerate(moms):
            mombuf[pl.ds(k * 16, 16)] = acc
        pltpu.sync_copy(pgblock, pg_out.at[pl.ds(b * (G * K * 3), G * K * 3)])
        pltpu.sync_copy(mombuf, mom_out.at[b])

    pg_flat, mom = k3(d2, ptsT, cxyz, t0)
    return pg_flat.reshape(B * G * K, 3), mom


_R = 1024
_NSTEPS = (B * NUM_GROUP * GROUP_SIZE) // _R
_NPTS = float(B * NUM_GROUP * GROUP_SIZE)


def _k4b_body(pg_ref, mom_ref, W1_ref, b1_ref, g1_ref, be1_ref,
              W2_ref, b2_ref, W3_ref, b3_ref, f3_ref, stats_ref, acc_ref):
    i = pl.program_id(0)

    @pl.when(i == 0)
    def _():
        acc_ref[...] = jnp.zeros_like(acc_ref)

    momr = mom_ref[...]
    s = [jnp.sum(momr[:, k * 16:(k + 1) * 16]) for k in range(9)]
    mx, my, mz = s[0] / _NPTS, s[1] / _NPTS, s[2] / _NPTS
    sxx, syy, szz = s[3] / _NPTS, s[4] / _NPTS, s[5] / _NPTS
    sxy, sxz, syz = s[6] / _NPTS, s[7] / _NPTS, s[8] / _NPTS

    w = W1_ref[...]
    wx, wy, wz = w[:, 0], w[:, 1], w[:, 2]
    lin = wx * mx + wy * my + wz * mz
    quad = (wx * wx * sxx + wy * wy * syy + wz * wz * szz
            + 2.0 * (wx * wy * sxy + wx * wz * sxz + wy * wz * syz))
    var1 = quad - lin * lin
    mu1 = lin + b1_ref[...]
    s1 = g1_ref[...] / jnp.sqrt(var1 + 1e-5)
    t1 = be1_ref[...] - mu1 * s1

    pg = pg_ref[...]
    f1 = lax.dot_general(pg, w, (((1,), (1,)), ((), ())),
                         preferred_element_type=jnp.float32) + b1_ref[...]
    f1 = jax.nn.relu(f1 * s1 + t1)
    f2 = lax.dot_general(f1, W2_ref[...], (((1,), (1,)), ((), ())),
                         preferred_element_type=jnp.float32) + b2_ref[...]
    g = f2.reshape(_R // GROUP_SIZE, GROUP_SIZE, 256)
    fg = jnp.max(g, axis=1, keepdims=True)
    h = jnp.concatenate([jnp.broadcast_to(fg, g.shape), g], axis=-1)
    h = h.reshape(_R, 512)
    f3 = lax.dot_general(h.astype(jnp.bfloat16),
                         W3_ref[...].astype(jnp.bfloat16),
                         (((1,), (1,)), ((), ())),
                         preferred_element_type=jnp.float32) + b3_ref[...]
    f3_ref[...] = f3.astype(jnp.bfloat16)
    acc_ref[0, :] += jnp.sum(f3, axis=0)
    acc_ref[1, :] += jnp.sum(f3 * f3, axis=0)
    stats_ref[...] = acc_ref[...]


def _k4c_body(f3_ref, stats_ref, g2_ref, be2_ref, W4_ref, b4_ref, tok_ref):
    mu = stats_ref[0, :] / _NPTS
    var = stats_ref[1, :] / _NPTS - mu * mu
    s2 = g2_ref[...] / jnp.sqrt(var + 1e-5)
    t2 = be2_ref[...] - mu * s2
    r = jax.nn.relu(f3_ref[...].astype(jnp.float32) * s2 + t2)
    f4 = lax.dot_general(r.astype(jnp.bfloat16),
                         W4_ref[...].astype(jnp.bfloat16),
                         (((1,), (1,)), ((), ())),
                         preferred_element_type=jnp.float32) + b4_ref[...]
    tok_ref[...] = jnp.max(f4.reshape(_R // GROUP_SIZE, GROUP_SIZE, ENC_DIM),
                           axis=1)


def _k4(pg_rows, mom, W1, b1, g1, be1, W2, b2, W3, b3, g2, be2, W4, b4):
    full = lambda s: pl.BlockSpec(s, lambda i: tuple(0 for _ in s))
    f3, _stats = pl.pallas_call(
        _k4b_body,
        grid=(_NSTEPS,),
        in_specs=[
            pl.BlockSpec((_R, 3), lambda i: (i, 0)),
            full((B, 144)), full((128, 3)), full((128,)), full((128,)),
            full((128,)), full((256, 128)), full((256,)),
            full((512, 512)), full((512,)),
        ],
        out_specs=[
            pl.BlockSpec((_R, 512), lambda i: (i, 0)),
            pl.BlockSpec((2, 512), lambda i: (0, 0)),
        ],
        out_shape=[
            jax.ShapeDtypeStruct((B * NUM_GROUP * GROUP_SIZE, 512),
                                 jnp.bfloat16),
            jax.ShapeDtypeStruct((2, 512), jnp.float32),
        ],
        scratch_shapes=[pltpu.VMEM((2, 512), jnp.float32)],
    )(pg_rows, mom, W1, b1, g1, be1, W2, b2, W3, b3)

    tokens = pl.pallas_call(
        _k4c_body,
        grid=(_NSTEPS,),
        in_specs=[
            pl.BlockSpec((_R, 512), lambda i: (i, 0)),
            full((2, 512)), full((512,)), full((512,)),
            full((ENC_DIM, 512)), full((ENC_DIM,)),
        ],
        out_specs=pl.BlockSpec((_R // GROUP_SIZE, ENC_DIM), lambda i: (i, 0)),
        out_shape=jax.ShapeDtypeStruct((B * NUM_GROUP, ENC_DIM), jnp.float32),
    )(f3, _stats, g2, be2, W4, b4)
    return tokens


def _k5_body(tok_ref, c_ref, Wp1_ref, bp1_ref, Wp2_ref, bp2_ref,
             ga1_ref, be1_ref, ga2_ref, be2_ref, clst_ref, clsp_ref, o_ref):
    G = NUM_GROUP
    c = c_ref[:, 0, 0, :]
    tok = tok_ref[0]
    ct = jnp.transpose(c)
    h1 = lax.dot_general(ct, Wp1_ref[...], (((1,), (1,)), ((), ())),
                         preferred_element_type=jnp.float32) + bp1_ref[...]
    pos = lax.dot_general(jax.nn.gelu(h1), Wp2_ref[...],
                          (((1,), (1,)), ((), ())),
                          preferred_element_type=jnp.float32) + bp2_ref[...]

    grid = jnp.floor(c * (1.0 / GRID_SIZE)).astype(jnp.int32)
    grid = grid - jnp.min(grid, axis=1, keepdims=True)
    grid = jnp.clip(grid, 0, (1 << BITS) - 1)

    def morton(x, y, z):
        code = jnp.zeros_like(x)
        for bb in range(BITS):
            code = (code
                    | (((x >> bb) & 1) << (3 * bb + 2))
                    | (((y >> bb) & 1) << (3 * bb + 1))
                    | (((z >> bb) & 1) << (3 * bb)))
        return code

    gx, gy, gz = grid[0:1], grid[1:2], grid[2:3]
    code_f = morton(gx, gy, gz)
    code_b = morton(gz, gy, gx)

    ii = lax.broadcasted_iota(jnp.int32, (G, G), 1)
    jj = lax.broadcasted_iota(jnp.int32, (G, G), 0)
    iota_d = jj

    def perm_matrix(code):
        cr = jnp.transpose(code)
        lt = code < cr
        eqm = jnp.logical_and(code == cr, ii < jj)
        rank = jnp.sum(jnp.logical_or(lt, eqm).astype(jnp.int32),
                       axis=1, keepdims=True)
        return (iota_d == jnp.transpose(rank)).astype(jnp.float32)

    Pf = perm_matrix(code_f)
    Pb = perm_matrix(code_b)

    def apply(P, m):
        return lax.dot_general(P, m, (((1,), (0,)), ((), ())),
                               preferred_element_type=jnp.float32)

    tok_f = apply(Pf, tok) * ga1_ref[...] + be1_ref[...]
    pos_f = apply(Pf, pos)
    tok_b = apply(Pb, tok) * ga2_ref[...] + be2_ref[...]
    pos_b = apply(Pb, pos)

    cls_row = clsp_ref[0, 0, :] + clst_ref[0, 0, :]
    xs = (tok_f + pos_f)
    xb = (tok_b + pos_b)
    total = jnp.sum(xs, axis=0) + jnp.sum(xb, axis=0) + cls_row
    mean = total * (1.0 / (2 * G + 1))
    o_ref[0, 0] = jnp.concatenate([cls_row, mean], axis=-1)


def _k5(tokens, cxyz, Wp1, bp1, Wp2, bp2, gamma1, beta1, gamma2, beta2,
        cls_token, cls_pos):
    G = NUM_GROUP
    full = lambda s: pl.BlockSpec(s, lambda b: tuple(0 for _ in s))
    out = pl.pallas_call(
        _k5_body,
        grid=(B,),
        in_specs=[
            pl.BlockSpec((1, G, TRANS_DIM), lambda b: (b, 0, 0)),
            pl.BlockSpec((3, 1, 1, G), lambda b: (0, b, 0, 0)),
            full((128, 3)), full((128,)), full((TRANS_DIM, 128)),
            full((TRANS_DIM,)),
            full((TRANS_DIM,)), full((TRANS_DIM,)),
            full((TRANS_DIM,)), full((TRANS_DIM,)),
            full((1, 1, TRANS_DIM)), full((1, 1, TRANS_DIM)),
        ],
        out_specs=pl.BlockSpec((1, 1, 2 * TRANS_DIM), lambda b: (b, 0, 0)),
        out_shape=jax.ShapeDtypeStruct((B, 1, 2 * TRANS_DIM), jnp.float32),
    )(tokens, cxyz.reshape(3, B, 1, G), Wp1, bp1, Wp2, bp2,
      gamma1, beta1, gamma2, beta2, cls_token, cls_pos)
    return out.reshape(B, 2 * TRANS_DIM)


def kernel(pts, W1, b1, g1, be1, W2, b2, W3, b3, g2, be2, W4, b4, Wp1, bp1, Wp2, bp2, gamma1, beta1, gamma2, beta2, cls_token, cls_pos):
    ptsT = jnp.transpose(pts, (2, 0, 1))
    cxyz = _k1(ptsT)
    d2, t0 = _k2(ptsT, cxyz)
    pg_rows, mom = _k3_sc(d2, ptsT, cxyz, t0)
    tokens = _k4(pg_rows, mom, W1, b1, g1, be1, W2, b2, W3, b3,
                 g2, be2, W4, b4).reshape(B, NUM_GROUP, ENC_DIM)
    return _k5(tokens, cxyz, Wp1, bp1, Wp2, bp2,
               gamma1, beta1, gamma2, beta2, cls_token, cls_pos)
